# Initial kernel scaffold; baseline (speedup 1.0000x reference)
import jax
import jax.numpy as jnp
from jax.experimental import pallas as pl


def kernel(z, frac_x, h, num_atoms, lengths, angles, edge_index, W_emb, b_emb, W_edge, b_edge, W_msg, b_msg, W_upd, b_upd, W_out_h, b_out_h, W_out_x, b_out_x):
    raise NotImplementedError("write your pallas kernel here")


# speedup vs baseline: 2.4551x; 2.4551x over previous
import functools

import jax
import jax.numpy as jnp
from jax import lax
from jax.experimental import pallas as pl
from jax.experimental.pallas import tpu as pltpu
from jax.experimental.pallas import tpu_sc as plsc

N = 10000
E = 320000
HID = 128
NRBF = 16
NOUT = 100
CUTOFF = 6.0
WX = HID + 16

NC = 2
NS = 16
NW = NC * NS
PER = E // NW
CHUNK = 80
NCHUNK = PER // CHUNK
NSTRIPE = N // NS


def _silu(v):
    return v * jax.nn.sigmoid(v)


def _dot(a, b):
    return jnp.dot(a, b, preferred_element_type=jnp.float32)


def _sc_gather2(table, src, dst):
    W = table.shape[1]
    mesh = plsc.VectorSubcoreMesh(core_axis_name="c", subcore_axis_name="s")

    @functools.partial(
        pl.kernel,
        out_type=[jax.ShapeDtypeStruct((E, W), jnp.float32),
                  jax.ShapeDtypeStruct((E, W), jnp.float32)],
        mesh=mesh,
        scratch_types=[pltpu.VMEM((CHUNK,), jnp.int32),
                       pltpu.VMEM((CHUNK, W), jnp.float32),
                       pltpu.SemaphoreType.DMA,
                       pltpu.VMEM_SHARED((N, W), jnp.float32)],
        compiler_params=pltpu.CompilerParams(use_tc_tiling_on_sc=False),
    )
    def k(tbl, s_idx, d_idx, o1, o2, idxv, rows, sem, shtbl):
        cid = lax.axis_index("c")
        sid = lax.axis_index("s")
        wid = sid * NC + cid

        @pl.when(sid == 0)
        def _stage():
            pltpu.sync_copy(tbl, shtbl)
        plsc.subcore_barrier()

        def run(idx_hbm, out_hbm):
            def body(i, carry):
                base = wid * PER + i * CHUNK
                pltpu.sync_copy(idx_hbm.at[pl.ds(base, CHUNK)], idxv)
                pltpu.async_copy(shtbl.at[idxv], rows, sem).wait()
                pltpu.sync_copy(rows, out_hbm.at[pl.ds(base, CHUNK)])
                return carry
            lax.fori_loop(0, NCHUNK, body, 0)

        run(s_idx, o1)
        run(d_idx, o2)

    return k(table, src, dst)


def _sc_scatter(rows_in, dst):
    W = rows_in.shape[1]
    mesh = plsc.VectorSubcoreMesh(core_axis_name="c", subcore_axis_name="s")
    zer = jnp.zeros((N, W), jnp.float32)

    @functools.partial(
        pl.kernel,
        out_type=jax.ShapeDtypeStruct((NC, N, W), jnp.float32),
        mesh=mesh,
        scratch_types=[pltpu.VMEM((CHUNK,), jnp.int32),
                       pltpu.VMEM((CHUNK, W), jnp.float32),
                       pltpu.VMEM_SHARED((N, W), jnp.float32)],
        compiler_params=pltpu.CompilerParams(use_tc_tiling_on_sc=False),
    )
    def k(rows_hbm, d_idx, zer_hbm, out, idxv, rows, agg):
        cid = lax.axis_index("c")
        sid = lax.axis_index("s")
        wid = sid * NC + cid

        @pl.when(sid == 0)
        def _zero():
            pltpu.sync_copy(zer_hbm, agg)
        plsc.subcore_barrier()

        def body(i, carry):
            base = wid * PER + i * CHUNK
            pltpu.sync_copy(d_idx.at[pl.ds(base, CHUNK)], idxv)
            pltpu.sync_copy(rows_hbm.at[pl.ds(base, CHUNK)], rows)
            pltpu.sync_copy(rows, agg.at[idxv], add=True)
            return carry
        lax.fori_loop(0, NCHUNK, body, 0)

        plsc.subcore_barrier()

        @pl.when(sid == 0)
        def _dump():
            pltpu.sync_copy(agg, out.at[cid])

    return k(rows_in, dst, zer)


BN = 2000
BE = 2000


def _tc_embed(h, zb, W_h, W_z, b):
    def body(h_ref, z_ref, wh_ref, wz_ref, b_ref, o_ref):
        acc = _dot(h_ref[...], wh_ref[...]) + _dot(z_ref[...], wz_ref[...]) + b_ref[...]
        o_ref[...] = _silu(acc)

    return pl.pallas_call(
        body,
        grid=(N // BN,),
        in_specs=[pl.BlockSpec((BN, 256), lambda i: (i, 0)),
                  pl.BlockSpec((BN, 256), lambda i: (i, 0)),
                  pl.BlockSpec((256, HID), lambda i: (0, 0)),
                  pl.BlockSpec((256, HID), lambda i: (0, 0)),
                  pl.BlockSpec((1, HID), lambda i: (0, 0))],
        out_specs=pl.BlockSpec((BN, HID), lambda i: (i, 0)),
        out_shape=jax.ShapeDtypeStruct((N, HID), jnp.float32),
    )(h, zb, W_h, W_z, b)


def _tc_pass1(Xs, Xd, We, be, W1, b1):
    def body(xs_ref, xd_ref, we_ref, be_ref, w1_ref, b1_ref,
             e1_ref, m1_ref, u_ref):
        xs = xs_ref[:, :HID]
        xd = xd_ref[:, :HID]
        cs = xs_ref[:, HID:]
        cd = xd_ref[:, HID:]
        dv = cd - cs
        dist = jnp.sqrt(jnp.sum(dv * dv, axis=1, keepdims=True) + 1e-9)
        centers = (CUTOFF / (NRBF - 1)) * lax.broadcasted_iota(
            jnp.int32, (1, NRBF), 1).astype(jnp.float32)
        diff = dist - centers
        rbf = jnp.exp(-10.0 * diff * diff)
        pre = (_dot(xs, we_ref[:HID]) + _dot(xd, we_ref[HID:2 * HID])
               + _dot(rbf, we_ref[2 * HID:]) + be_ref[...])
        e0 = _silu(pre)
        t = e0 + xs + xd
        m1 = _silu(_dot(t, w1_ref[...]) + b1_ref[...])
        e1_ref[...] = e0 + m1
        m1_ref[...] = m1
        u_ref[...] = dv / dist

    return pl.pallas_call(
        body,
        grid=(E // BE,),
        in_specs=[pl.BlockSpec((BE, WX), lambda i: (i, 0)),
                  pl.BlockSpec((BE, WX), lambda i: (i, 0)),
                  pl.BlockSpec((2 * HID + NRBF, HID), lambda i: (0, 0)),
                  pl.BlockSpec((1, HID), lambda i: (0, 0)),
                  pl.BlockSpec((HID, HID), lambda i: (0, 0)),
                  pl.BlockSpec((1, HID), lambda i: (0, 0))],
        out_specs=[pl.BlockSpec((BE, HID), lambda i: (i, 0)),
                   pl.BlockSpec((BE, HID), lambda i: (i, 0)),
                   pl.BlockSpec((BE, 16), lambda i: (i, 0))],
        out_shape=[jax.ShapeDtypeStruct((E, HID), jnp.float32),
                   jax.ShapeDtypeStruct((E, HID), jnp.float32),
                   jax.ShapeDtypeStruct((E, 16), jnp.float32)],
    )(Xs, Xd, We, be, W1, b1)


def _tc_pass2(e, Xs, Xd, W, b):
    def body(e_ref, xs_ref, xd_ref, w_ref, b_ref, e2_ref, m_ref):
        t = e_ref[...] + xs_ref[...] + xd_ref[...]
        m = _silu(_dot(t, w_ref[...]) + b_ref[...])
        m_ref[...] = m
        e2_ref[...] = e_ref[...] + m

    return pl.pallas_call(
        body,
        grid=(E // BE,),
        in_specs=[pl.BlockSpec((BE, HID), lambda i: (i, 0)),
                  pl.BlockSpec((BE, HID), lambda i: (i, 0)),
                  pl.BlockSpec((BE, HID), lambda i: (i, 0)),
                  pl.BlockSpec((HID, HID), lambda i: (0, 0)),
                  pl.BlockSpec((1, HID), lambda i: (0, 0))],
        out_specs=[pl.BlockSpec((BE, HID), lambda i: (i, 0)),
                   pl.BlockSpec((BE, HID), lambda i: (i, 0))],
        out_shape=[jax.ShapeDtypeStruct((E, HID), jnp.float32),
                   jax.ShapeDtypeStruct((E, HID), jnp.float32)],
    )(e, Xs, Xd, W, b)


def _tc_pass3(e, Xs, Xd, unitp, W, b, wox8, box8):
    def body(e_ref, xs_ref, xd_ref, u_ref, w_ref, b_ref, wox_ref, box_ref,
             o_ref):
        t = e_ref[...] + xs_ref[...] + xd_ref[...]
        m = _silu(_dot(t, w_ref[...]) + b_ref[...])
        e3 = e_ref[...] + m
        scal = _dot(e3, wox_ref[...]) + box_ref[...]
        o_ref[:, :HID] = m
        o_ref[:, HID:] = scal[:, 0:1] * u_ref[...]

    return pl.pallas_call(
        body,
        grid=(E // BE,),
        in_specs=[pl.BlockSpec((BE, HID), lambda i: (i, 0)),
                  pl.BlockSpec((BE, HID), lambda i: (i, 0)),
                  pl.BlockSpec((BE, HID), lambda i: (i, 0)),
                  pl.BlockSpec((BE, 16), lambda i: (i, 0)),
                  pl.BlockSpec((HID, HID), lambda i: (0, 0)),
                  pl.BlockSpec((1, HID), lambda i: (0, 0)),
                  pl.BlockSpec((HID, 8), lambda i: (0, 0)),
                  pl.BlockSpec((1, 8), lambda i: (0, 0))],
        out_specs=pl.BlockSpec((BE, WX), lambda i: (i, 0)),
        out_shape=jax.ShapeDtypeStruct((E, WX), jnp.float32),
    )(e, Xs, Xd, unitp, W, b, wox8, box8)


def _tc_node(x, a0, a1, W, b):
    def body(x_ref, a0_ref, a1_ref, w_ref, b_ref, o_ref):
        agg = a0_ref[...] + a1_ref[...]
        o_ref[...] = x_ref[...] + _silu(_dot(agg, w_ref[...]) + b_ref[...])

    return pl.pallas_call(
        body,
        grid=(N // BN,),
        in_specs=[pl.BlockSpec((BN, HID), lambda i: (i, 0)),
                  pl.BlockSpec((BN, HID), lambda i: (i, 0)),
                  pl.BlockSpec((BN, HID), lambda i: (i, 0)),
                  pl.BlockSpec((HID, HID), lambda i: (0, 0)),
                  pl.BlockSpec((1, HID), lambda i: (0, 0))],
        out_specs=pl.BlockSpec((BN, HID), lambda i: (i, 0)),
        out_shape=jax.ShapeDtypeStruct((N, HID), jnp.float32),
    )(x, a0, a1, W, b)


def _tc_node3(x, a0, a1, W, b):
    def body(x_ref, a0_ref, a1_ref, w_ref, b_ref, x3_ref, eps_ref):
        s = a0_ref[...] + a1_ref[...]
        agg = s[:, :HID]
        x3_ref[...] = x_ref[...] + _silu(_dot(agg, w_ref[...]) + b_ref[...])
        eps_ref[...] = s[:, HID:]

    return pl.pallas_call(
        body,
        grid=(N // BN,),
        in_specs=[pl.BlockSpec((BN, HID), lambda i: (i, 0)),
                  pl.BlockSpec((BN, WX), lambda i: (i, 0)),
                  pl.BlockSpec((BN, WX), lambda i: (i, 0)),
                  pl.BlockSpec((HID, HID), lambda i: (0, 0)),
                  pl.BlockSpec((1, HID), lambda i: (0, 0))],
        out_specs=[pl.BlockSpec((BN, HID), lambda i: (i, 0)),
                   pl.BlockSpec((BN, 16), lambda i: (i, 0))],
        out_shape=[jax.ShapeDtypeStruct((N, HID), jnp.float32),
                   jax.ShapeDtypeStruct((N, 16), jnp.float32)],
    )(x, a0, a1, W, b)


def _tc_outh(x, W, b):
    def body(x_ref, w_ref, b_ref, o_ref):
        o_ref[...] = _dot(x_ref[...], w_ref[...]) + b_ref[...]

    return pl.pallas_call(
        body,
        grid=(N // BN,),
        in_specs=[pl.BlockSpec((BN, HID), lambda i: (i, 0)),
                  pl.BlockSpec((HID, NOUT), lambda i: (0, 0)),
                  pl.BlockSpec((1, NOUT), lambda i: (0, 0))],
        out_specs=pl.BlockSpec((BN, NOUT), lambda i: (i, 0)),
        out_shape=jax.ShapeDtypeStruct((N, NOUT), jnp.float32),
    )(x, W, b)


def _lattice(lengths, angles):
    a, b, c = lengths[:, 0], lengths[:, 1], lengths[:, 2]
    al = jnp.deg2rad(angles[:, 0])
    be = jnp.deg2rad(angles[:, 1])
    ga = jnp.deg2rad(angles[:, 2])
    cos_a, cos_b, cos_g = jnp.cos(al), jnp.cos(be), jnp.cos(ga)
    sin_g = jnp.sin(ga)
    zeros = jnp.zeros_like(a)
    v1 = jnp.stack([a, zeros, zeros], -1)
    v2 = jnp.stack([b * cos_g, b * sin_g, zeros], -1)
    cx = cos_b
    cy = (cos_a - cos_b * cos_g) / sin_g
    cz = jnp.sqrt(jnp.clip(1.0 - cx ** 2 - cy ** 2, 1e-8, None))
    v3 = jnp.stack([c * cx, c * cy, c * cz], -1)
    return jnp.stack([v1, v2, v3], 1)


def kernel(z, frac_x, h, num_atoms, lengths, angles, edge_index,
           W_emb, b_emb, W_edge, b_edge, W_msg, b_msg, W_upd, b_upd,
           W_out_h, b_out_h, W_out_x, b_out_x):
    src = edge_index[0]
    dst = edge_index[1]

    zb = jnp.repeat(z, num_atoms, axis=0, total_repeat_length=N)
    lat = _lattice(lengths, angles)
    latb = jnp.repeat(lat, num_atoms, axis=0, total_repeat_length=N)
    cart = jnp.einsum('ni,nij->nj', frac_x, latb)
    cart_pad = jnp.pad(cart, ((0, 0), (0, 13)))

    x0 = _tc_embed(h, zb, W_emb[:256], W_emb[256:], b_emb.reshape(1, -1))
    tbl1 = jnp.concatenate([x0, cart_pad], axis=1)

    Xs1, Xd1 = _sc_gather2(tbl1, src, dst)
    e1, m1, unitp = _tc_pass1(Xs1, Xd1, W_edge, b_edge.reshape(1, -1),
                              W_msg[0], b_msg[0].reshape(1, -1))
    P1 = _sc_scatter(m1, dst)
    x1 = _tc_node(x0, P1[0], P1[1], W_upd[0], b_upd[0].reshape(1, -1))

    Xs2, Xd2 = _sc_gather2(x1, src, dst)
    e2, m2 = _tc_pass2(e1, Xs2, Xd2, W_msg[1], b_msg[1].reshape(1, -1))
    P2 = _sc_scatter(m2, dst)
    x2 = _tc_node(x1, P2[0], P2[1], W_upd[1], b_upd[1].reshape(1, -1))

    wox8 = jnp.pad(W_out_x, ((0, 0), (0, 7)))
    box8 = jnp.pad(b_out_x, (0, 7)).reshape(1, 8)
    Xs3, Xd3 = _sc_gather2(x2, src, dst)
    mm3 = _tc_pass3(e2, Xs3, Xd3, unitp, W_msg[2], b_msg[2].reshape(1, -1),
                    wox8, box8)
    P3 = _sc_scatter(mm3, dst)
    x3, epsp = _tc_node3(x2, P3[0], P3[1], W_upd[2], b_upd[2].reshape(1, -1))

    pred_eps_h = _tc_outh(x3, W_out_h, b_out_h.reshape(1, -1))
    pred_eps_x = epsp[:, :3]
    return (pred_eps_x, pred_eps_h)


# speedup vs baseline: 2.9908x; 1.2182x over previous
import functools

import jax
import jax.numpy as jnp
from jax import lax
from jax.experimental import pallas as pl
from jax.experimental.pallas import tpu as pltpu
from jax.experimental.pallas import tpu_sc as plsc

N = 10000
E = 320000
HID = 128
NRBF = 16
NOUT = 100
CUTOFF = 6.0
WX = HID + 16

NC = 2
NS = 16
NW = NC * NS
PER = E // NW
CHUNK = 40
NCHUNK = PER // CHUNK
K = 5
NGROUP = NCHUNK // K


def _silu(v):
    return v * jax.nn.sigmoid(v)


def _dot(a, b):
    return jnp.dot(a, b, preferred_element_type=jnp.float32)


def _sc_gather2(table, src, dst):
    W = table.shape[1]
    mesh = plsc.VectorSubcoreMesh(core_axis_name="c", subcore_axis_name="s")

    @functools.partial(
        pl.kernel,
        out_type=[jax.ShapeDtypeStruct((E, W), jnp.float32),
                  jax.ShapeDtypeStruct((E, W), jnp.float32)],
        mesh=mesh,
        scratch_types=([pltpu.VMEM((NCHUNK, CHUNK), jnp.int32)]
                       + [pltpu.VMEM((CHUNK, W), jnp.float32)
                          for _ in range(K)]
                       + [pltpu.SemaphoreType.DMA, pltpu.SemaphoreType.DMA,
                          pltpu.VMEM_SHARED((N, W), jnp.float32)]),
        compiler_params=pltpu.CompilerParams(use_tc_tiling_on_sc=False),
    )
    def k(tbl, s_idx, d_idx, o1, o2, idx2, r0, r1, r2, r3, r4,
          gsem, ssem, shtbl):
        cid = lax.axis_index("c")
        sid = lax.axis_index("s")
        wid = sid * NC + cid
        bufs = [r0, r1, r2, r3, r4]

        @pl.when(sid == 0)
        def _stage():
            pltpu.sync_copy(tbl, shtbl)
        plsc.subcore_barrier()

        def run(idx3_hbm, out_hbm):
            pltpu.sync_copy(idx3_hbm.at[wid], idx2)

            def group(g, carry):
                descs = [pltpu.async_copy(
                    shtbl.at[idx2.at[g * K + b]], bufs[b], gsem)
                    for b in range(K)]
                for d in descs:
                    d.wait()
                base = wid * PER + g * (K * CHUNK)
                descs = [pltpu.async_copy(
                    bufs[b], out_hbm.at[pl.ds(base + b * CHUNK, CHUNK)], ssem)
                    for b in range(K)]
                for d in descs:
                    d.wait()
                return carry
            lax.fori_loop(0, NGROUP, group, 0)

        run(s_idx, o1)
        run(d_idx, o2)

    src3 = src.reshape(NW, NCHUNK, CHUNK)
    dst3 = dst.reshape(NW, NCHUNK, CHUNK)
    return k(table, src3, dst3)


def _sc_scatter(rows_in, dst):
    W = rows_in.shape[1]
    mesh = plsc.VectorSubcoreMesh(core_axis_name="c", subcore_axis_name="s")
    zer = jnp.zeros((N, W), jnp.float32)

    @functools.partial(
        pl.kernel,
        out_type=jax.ShapeDtypeStruct((NC, N, W), jnp.float32),
        mesh=mesh,
        scratch_types=([pltpu.VMEM((NCHUNK, CHUNK), jnp.int32)]
                       + [pltpu.VMEM((CHUNK, W), jnp.float32)
                          for _ in range(K)]
                       + [pltpu.SemaphoreType.DMA, pltpu.SemaphoreType.DMA,
                          pltpu.VMEM_SHARED((N, W), jnp.float32)]),
        compiler_params=pltpu.CompilerParams(use_tc_tiling_on_sc=False),
    )
    def k(rows_hbm, d_idx, zer_hbm, out, idx2, r0, r1, r2, r3, r4,
          lsem, asem, agg):
        cid = lax.axis_index("c")
        sid = lax.axis_index("s")
        wid = sid * NC + cid
        bufs = [r0, r1, r2, r3, r4]

        @pl.when(sid == 0)
        def _zero():
            pltpu.sync_copy(zer_hbm, agg)
        pltpu.sync_copy(d_idx.at[wid], idx2)
        plsc.subcore_barrier()

        def group(g, carry):
            base = wid * PER + g * (K * CHUNK)
            descs = [pltpu.async_copy(
                rows_hbm.at[pl.ds(base + b * CHUNK, CHUNK)], bufs[b], lsem)
                for b in range(K)]
            for d in descs:
                d.wait()
            descs = [pltpu.async_copy(
                bufs[b], agg.at[idx2.at[g * K + b]], asem, add=True)
                for b in range(K)]
            for d in descs:
                d.wait()
            return carry
        lax.fori_loop(0, NGROUP, group, 0)

        plsc.subcore_barrier()

        @pl.when(sid == 0)
        def _dump():
            pltpu.sync_copy(agg, out.at[cid])

    dst3 = dst.reshape(NW, NCHUNK, CHUNK)
    return k(rows_in, dst3, zer)


BN = 2000
BE = 2000


def _tc_embed(h, zb, W_h, W_z, b):
    def body(h_ref, z_ref, wh_ref, wz_ref, b_ref, o_ref):
        acc = _dot(h_ref[...], wh_ref[...]) + _dot(z_ref[...], wz_ref[...]) + b_ref[...]
        o_ref[...] = _silu(acc)

    return pl.pallas_call(
        body,
        grid=(N // BN,),
        in_specs=[pl.BlockSpec((BN, 256), lambda i: (i, 0)),
                  pl.BlockSpec((BN, 256), lambda i: (i, 0)),
                  pl.BlockSpec((256, HID), lambda i: (0, 0)),
                  pl.BlockSpec((256, HID), lambda i: (0, 0)),
                  pl.BlockSpec((1, HID), lambda i: (0, 0))],
        out_specs=pl.BlockSpec((BN, HID), lambda i: (i, 0)),
        out_shape=jax.ShapeDtypeStruct((N, HID), jnp.float32),
    )(h, zb, W_h, W_z, b)


def _tc_pass1(Xs, Xd, We, be, W1, b1):
    def body(xs_ref, xd_ref, we_ref, be_ref, w1_ref, b1_ref,
             e1_ref, m1_ref, u_ref):
        xs = xs_ref[:, :HID]
        xd = xd_ref[:, :HID]
        cs = xs_ref[:, HID:]
        cd = xd_ref[:, HID:]
        dv = cd - cs
        dist = jnp.sqrt(jnp.sum(dv * dv, axis=1, keepdims=True) + 1e-9)
        centers = (CUTOFF / (NRBF - 1)) * lax.broadcasted_iota(
            jnp.int32, (1, NRBF), 1).astype(jnp.float32)
        diff = dist - centers
        rbf = jnp.exp(-10.0 * diff * diff)
        pre = (_dot(xs, we_ref[:HID]) + _dot(xd, we_ref[HID:2 * HID])
               + _dot(rbf, we_ref[2 * HID:]) + be_ref[...])
        e0 = _silu(pre)
        t = e0 + xs + xd
        m1 = _silu(_dot(t, w1_ref[...]) + b1_ref[...])
        e1_ref[...] = e0 + m1
        m1_ref[...] = m1
        u_ref[...] = dv / dist

    return pl.pallas_call(
        body,
        grid=(E // BE,),
        in_specs=[pl.BlockSpec((BE, WX), lambda i: (i, 0)),
                  pl.BlockSpec((BE, WX), lambda i: (i, 0)),
                  pl.BlockSpec((2 * HID + NRBF, HID), lambda i: (0, 0)),
                  pl.BlockSpec((1, HID), lambda i: (0, 0)),
                  pl.BlockSpec((HID, HID), lambda i: (0, 0)),
                  pl.BlockSpec((1, HID), lambda i: (0, 0))],
        out_specs=[pl.BlockSpec((BE, HID), lambda i: (i, 0)),
                   pl.BlockSpec((BE, HID), lambda i: (i, 0)),
                   pl.BlockSpec((BE, 16), lambda i: (i, 0))],
        out_shape=[jax.ShapeDtypeStruct((E, HID), jnp.float32),
                   jax.ShapeDtypeStruct((E, HID), jnp.float32),
                   jax.ShapeDtypeStruct((E, 16), jnp.float32)],
    )(Xs, Xd, We, be, W1, b1)


def _tc_pass2(e, Xs, Xd, W, b):
    def body(e_ref, xs_ref, xd_ref, w_ref, b_ref, e2_ref, m_ref):
        t = e_ref[...] + xs_ref[...] + xd_ref[...]
        m = _silu(_dot(t, w_ref[...]) + b_ref[...])
        m_ref[...] = m
        e2_ref[...] = e_ref[...] + m

    return pl.pallas_call(
        body,
        grid=(E // BE,),
        in_specs=[pl.BlockSpec((BE, HID), lambda i: (i, 0)),
                  pl.BlockSpec((BE, HID), lambda i: (i, 0)),
                  pl.BlockSpec((BE, HID), lambda i: (i, 0)),
                  pl.BlockSpec((HID, HID), lambda i: (0, 0)),
                  pl.BlockSpec((1, HID), lambda i: (0, 0))],
        out_specs=[pl.BlockSpec((BE, HID), lambda i: (i, 0)),
                   pl.BlockSpec((BE, HID), lambda i: (i, 0))],
        out_shape=[jax.ShapeDtypeStruct((E, HID), jnp.float32),
                   jax.ShapeDtypeStruct((E, HID), jnp.float32)],
    )(e, Xs, Xd, W, b)


def _tc_pass3(e, Xs, Xd, unitp, W, b, wox8, box8):
    def body(e_ref, xs_ref, xd_ref, u_ref, w_ref, b_ref, wox_ref, box_ref,
             o_ref):
        t = e_ref[...] + xs_ref[...] + xd_ref[...]
        m = _silu(_dot(t, w_ref[...]) + b_ref[...])
        e3 = e_ref[...] + m
        scal = _dot(e3, wox_ref[...]) + box_ref[...]
        o_ref[:, :HID] = m
        o_ref[:, HID:] = scal[:, 0:1] * u_ref[...]

    return pl.pallas_call(
        body,
        grid=(E // BE,),
        in_specs=[pl.BlockSpec((BE, HID), lambda i: (i, 0)),
                  pl.BlockSpec((BE, HID), lambda i: (i, 0)),
                  pl.BlockSpec((BE, HID), lambda i: (i, 0)),
                  pl.BlockSpec((BE, 16), lambda i: (i, 0)),
                  pl.BlockSpec((HID, HID), lambda i: (0, 0)),
                  pl.BlockSpec((1, HID), lambda i: (0, 0)),
                  pl.BlockSpec((HID, 8), lambda i: (0, 0)),
                  pl.BlockSpec((1, 8), lambda i: (0, 0))],
        out_specs=pl.BlockSpec((BE, WX), lambda i: (i, 0)),
        out_shape=jax.ShapeDtypeStruct((E, WX), jnp.float32),
    )(e, Xs, Xd, unitp, W, b, wox8, box8)


def _tc_node(x, a0, a1, W, b):
    def body(x_ref, a0_ref, a1_ref, w_ref, b_ref, o_ref):
        agg = a0_ref[...] + a1_ref[...]
        o_ref[...] = x_ref[...] + _silu(_dot(agg, w_ref[...]) + b_ref[...])

    return pl.pallas_call(
        body,
        grid=(N // BN,),
        in_specs=[pl.BlockSpec((BN, HID), lambda i: (i, 0)),
                  pl.BlockSpec((BN, HID), lambda i: (i, 0)),
                  pl.BlockSpec((BN, HID), lambda i: (i, 0)),
                  pl.BlockSpec((HID, HID), lambda i: (0, 0)),
                  pl.BlockSpec((1, HID), lambda i: (0, 0))],
        out_specs=pl.BlockSpec((BN, HID), lambda i: (i, 0)),
        out_shape=jax.ShapeDtypeStruct((N, HID), jnp.float32),
    )(x, a0, a1, W, b)


def _tc_node3(x, a0, a1, W, b):
    def body(x_ref, a0_ref, a1_ref, w_ref, b_ref, x3_ref, eps_ref):
        s = a0_ref[...] + a1_ref[...]
        agg = s[:, :HID]
        x3_ref[...] = x_ref[...] + _silu(_dot(agg, w_ref[...]) + b_ref[...])
        eps_ref[...] = s[:, HID:]

    return pl.pallas_call(
        body,
        grid=(N // BN,),
        in_specs=[pl.BlockSpec((BN, HID), lambda i: (i, 0)),
                  pl.BlockSpec((BN, WX), lambda i: (i, 0)),
                  pl.BlockSpec((BN, WX), lambda i: (i, 0)),
                  pl.BlockSpec((HID, HID), lambda i: (0, 0)),
                  pl.BlockSpec((1, HID), lambda i: (0, 0))],
        out_specs=[pl.BlockSpec((BN, HID), lambda i: (i, 0)),
                   pl.BlockSpec((BN, 16), lambda i: (i, 0))],
        out_shape=[jax.ShapeDtypeStruct((N, HID), jnp.float32),
                   jax.ShapeDtypeStruct((N, 16), jnp.float32)],
    )(x, a0, a1, W, b)


def _tc_outh(x, W, b):
    def body(x_ref, w_ref, b_ref, o_ref):
        o_ref[...] = _dot(x_ref[...], w_ref[...]) + b_ref[...]

    return pl.pallas_call(
        body,
        grid=(N // BN,),
        in_specs=[pl.BlockSpec((BN, HID), lambda i: (i, 0)),
                  pl.BlockSpec((HID, NOUT), lambda i: (0, 0)),
                  pl.BlockSpec((1, NOUT), lambda i: (0, 0))],
        out_specs=pl.BlockSpec((BN, NOUT), lambda i: (i, 0)),
        out_shape=jax.ShapeDtypeStruct((N, NOUT), jnp.float32),
    )(x, W, b)


def _lattice(lengths, angles):
    a, b, c = lengths[:, 0], lengths[:, 1], lengths[:, 2]
    al = jnp.deg2rad(angles[:, 0])
    be = jnp.deg2rad(angles[:, 1])
    ga = jnp.deg2rad(angles[:, 2])
    cos_a, cos_b, cos_g = jnp.cos(al), jnp.cos(be), jnp.cos(ga)
    sin_g = jnp.sin(ga)
    zeros = jnp.zeros_like(a)
    v1 = jnp.stack([a, zeros, zeros], -1)
    v2 = jnp.stack([b * cos_g, b * sin_g, zeros], -1)
    cx = cos_b
    cy = (cos_a - cos_b * cos_g) / sin_g
    cz = jnp.sqrt(jnp.clip(1.0 - cx ** 2 - cy ** 2, 1e-8, None))
    v3 = jnp.stack([c * cx, c * cy, c * cz], -1)
    return jnp.stack([v1, v2, v3], 1)


def kernel(z, frac_x, h, num_atoms, lengths, angles, edge_index,
           W_emb, b_emb, W_edge, b_edge, W_msg, b_msg, W_upd, b_upd,
           W_out_h, b_out_h, W_out_x, b_out_x):
    src = edge_index[0]
    dst = edge_index[1]

    zb = jnp.repeat(z, num_atoms, axis=0, total_repeat_length=N)
    lat = _lattice(lengths, angles)
    latb = jnp.repeat(lat, num_atoms, axis=0, total_repeat_length=N)
    cart = jnp.einsum('ni,nij->nj', frac_x, latb)
    cart_pad = jnp.pad(cart, ((0, 0), (0, 13)))

    x0 = _tc_embed(h, zb, W_emb[:256], W_emb[256:], b_emb.reshape(1, -1))
    tbl1 = jnp.concatenate([x0, cart_pad], axis=1)

    Xs1, Xd1 = _sc_gather2(tbl1, src, dst)
    e1, m1, unitp = _tc_pass1(Xs1, Xd1, W_edge, b_edge.reshape(1, -1),
                              W_msg[0], b_msg[0].reshape(1, -1))
    P1 = _sc_scatter(m1, dst)
    x1 = _tc_node(x0, P1[0], P1[1], W_upd[0], b_upd[0].reshape(1, -1))

    Xs2, Xd2 = _sc_gather2(x1, src, dst)
    e2, m2 = _tc_pass2(e1, Xs2, Xd2, W_msg[1], b_msg[1].reshape(1, -1))
    P2 = _sc_scatter(m2, dst)
    x2 = _tc_node(x1, P2[0], P2[1], W_upd[1], b_upd[1].reshape(1, -1))

    wox8 = jnp.pad(W_out_x, ((0, 0), (0, 7)))
    box8 = jnp.pad(b_out_x, (0, 7)).reshape(1, 8)
    Xs3, Xd3 = _sc_gather2(x2, src, dst)
    mm3 = _tc_pass3(e2, Xs3, Xd3, unitp, W_msg[2], b_msg[2].reshape(1, -1),
                    wox8, box8)
    P3 = _sc_scatter(mm3, dst)
    x3, epsp = _tc_node3(x2, P3[0], P3[1], W_upd[2], b_upd[2].reshape(1, -1))

    pred_eps_h = _tc_outh(x3, W_out_h, b_out_h.reshape(1, -1))
    pred_eps_x = epsp[:, :3]
    return (pred_eps_x, pred_eps_h)


# speedup vs baseline: 3.9546x; 1.3223x over previous
import functools

import jax
import jax.numpy as jnp
from jax import lax
from jax.experimental import pallas as pl
from jax.experimental.pallas import tpu as pltpu
from jax.experimental.pallas import tpu_sc as plsc

N = 10000
E = 320000
HID = 128
NRBF = 16
NOUT = 100
CUTOFF = 6.0
WX = HID + 16

NC = 2
NS = 16
NW = NC * NS
PER = E // NW
CHUNK = 40
NCHUNK = PER // CHUNK
K = 5
NGROUP = NCHUNK // K


def _silu(v):
    return v * jax.nn.sigmoid(v)


def _dot(a, b):
    return jnp.dot(a, b, preferred_element_type=jnp.float32)


def _sc_gather2(table, src3, dst3, tiled=True):
    W = table.shape[1]
    mesh = plsc.VectorSubcoreMesh(core_axis_name="c", subcore_axis_name="s")
    params = (pltpu.CompilerParams() if tiled
              else pltpu.CompilerParams(use_tc_tiling_on_sc=False))

    @functools.partial(
        pl.kernel,
        out_type=[jax.ShapeDtypeStruct((E, W), jnp.float32),
                  jax.ShapeDtypeStruct((E, W), jnp.float32)],
        mesh=mesh,
        scratch_types=([pltpu.VMEM((PER,), jnp.int32)]
                       + [pltpu.VMEM((CHUNK, W), jnp.float32)
                          for _ in range(K)]
                       + [pltpu.SemaphoreType.DMA, pltpu.SemaphoreType.DMA,
                          pltpu.VMEM_SHARED((N, W), jnp.float32)]),
        compiler_params=params,
    )
    def k(tbl, s_idx, d_idx, o1, o2, idx1, r0, r1, r2, r3, r4,
          gsem, ssem, shtbl):
        cid = lax.axis_index("c")
        sid = lax.axis_index("s")
        wid = sid * NC + cid
        bufs = [r0, r1, r2, r3, r4]

        @pl.when(sid == 0)
        def _stage():
            pltpu.sync_copy(tbl, shtbl)
        plsc.subcore_barrier()

        def run(idx2_hbm, out_hbm):
            pltpu.sync_copy(idx2_hbm.at[wid], idx1)

            def group(g, carry):
                descs = [pltpu.async_copy(
                    shtbl.at[idx1.at[pl.ds((g * K + b) * CHUNK, CHUNK)]],
                    bufs[b], gsem)
                    for b in range(K)]
                for d in descs:
                    d.wait()
                base = wid * PER + g * (K * CHUNK)
                descs = [pltpu.async_copy(
                    bufs[b], out_hbm.at[pl.ds(base + b * CHUNK, CHUNK)], ssem)
                    for b in range(K)]
                for d in descs:
                    d.wait()
                return carry
            lax.fori_loop(0, NGROUP, group, 0)

        run(s_idx, o1)
        run(d_idx, o2)

    return k(table, src3.reshape(NW, PER), dst3.reshape(NW, PER))


def _sc_scatter(rows_in, dst3, tiled=True):
    W = rows_in.shape[1]
    mesh = plsc.VectorSubcoreMesh(core_axis_name="c", subcore_axis_name="s")
    zer = jnp.zeros((N, W), jnp.float32)
    params = (pltpu.CompilerParams() if tiled
              else pltpu.CompilerParams(use_tc_tiling_on_sc=False))

    @functools.partial(
        pl.kernel,
        out_type=jax.ShapeDtypeStruct((NC, N, W), jnp.float32),
        mesh=mesh,
        scratch_types=([pltpu.VMEM((K, CHUNK), jnp.int32)]
                       + [pltpu.VMEM((CHUNK, W), jnp.float32)
                          for _ in range(K)]
                       + [pltpu.SemaphoreType.DMA, pltpu.SemaphoreType.DMA,
                          pltpu.SemaphoreType.DMA,
                          pltpu.VMEM_SHARED((N, W), jnp.float32)]),
        compiler_params=params,
    )
    def k(rows_hbm, d_idx, zer_hbm, out, idxb, r0, r1, r2, r3, r4,
          isem, lsem, asem, agg):
        cid = lax.axis_index("c")
        sid = lax.axis_index("s")
        wid = sid * NC + cid
        bufs = [r0, r1, r2, r3, r4]

        @pl.when(sid == 0)
        def _zero():
            pltpu.sync_copy(zer_hbm, agg)
        plsc.subcore_barrier()

        def group(g, carry):
            base = wid * PER + g * (K * CHUNK)
            idescs = [pltpu.async_copy(
                d_idx.at[wid, g * K + b], idxb.at[b], isem)
                for b in range(K)]
            ldescs = [pltpu.async_copy(
                rows_hbm.at[pl.ds(base + b * CHUNK, CHUNK)], bufs[b], lsem)
                for b in range(K)]
            for d in idescs:
                d.wait()
            for d in ldescs:
                d.wait()
            descs = [pltpu.async_copy(
                bufs[b], agg.at[idxb.at[b]], asem, add=True)
                for b in range(K)]
            for d in descs:
                d.wait()
            return carry
        lax.fori_loop(0, NGROUP, group, 0)

        plsc.subcore_barrier()

        @pl.when(sid == 0)
        def _dump():
            pltpu.sync_copy(agg, out.at[cid])

    return k(rows_in, dst3, zer)


BN = 2000
BE = 2000


def _tc_embed(h, zb, W_h, W_z, b):
    def body(h_ref, z_ref, wh_ref, wz_ref, b_ref, o_ref):
        acc = _dot(h_ref[...], wh_ref[...]) + _dot(z_ref[...], wz_ref[...]) + b_ref[...]
        o_ref[...] = _silu(acc)

    return pl.pallas_call(
        body,
        grid=(N // BN,),
        in_specs=[pl.BlockSpec((BN, 256), lambda i: (i, 0)),
                  pl.BlockSpec((BN, 256), lambda i: (i, 0)),
                  pl.BlockSpec((256, HID), lambda i: (0, 0)),
                  pl.BlockSpec((256, HID), lambda i: (0, 0)),
                  pl.BlockSpec((1, HID), lambda i: (0, 0))],
        out_specs=pl.BlockSpec((BN, HID), lambda i: (i, 0)),
        out_shape=jax.ShapeDtypeStruct((N, HID), jnp.float32),
    )(h, zb, W_h, W_z, b)


def _tc_pass1(Xs, Xd, Cs, Cd, We, be, W1, b1):
    def body(xs_ref, xd_ref, cs_ref, cd_ref, we_ref, be_ref, w1_ref, b1_ref,
             e1_ref, m1_ref, u_ref):
        xs = xs_ref[...]
        xd = xd_ref[...]
        dv = cd_ref[...] - cs_ref[...]
        dist = jnp.sqrt(jnp.sum(dv * dv, axis=1, keepdims=True) + 1e-9)
        centers = (CUTOFF / (NRBF - 1)) * lax.broadcasted_iota(
            jnp.int32, (1, NRBF), 1).astype(jnp.float32)
        diff = dist - centers
        rbf = jnp.exp(-10.0 * diff * diff)
        pre = (_dot(xs, we_ref[:HID]) + _dot(xd, we_ref[HID:2 * HID])
               + _dot(rbf, we_ref[2 * HID:]) + be_ref[...])
        e0 = _silu(pre)
        t = e0 + xs + xd
        m1 = _silu(_dot(t, w1_ref[...]) + b1_ref[...])
        e1_ref[...] = e0 + m1
        m1_ref[...] = m1
        u_ref[...] = dv / dist

    return pl.pallas_call(
        body,
        grid=(E // BE,),
        in_specs=[pl.BlockSpec((BE, HID), lambda i: (i, 0)),
                  pl.BlockSpec((BE, HID), lambda i: (i, 0)),
                  pl.BlockSpec((BE, 16), lambda i: (i, 0)),
                  pl.BlockSpec((BE, 16), lambda i: (i, 0)),
                  pl.BlockSpec((2 * HID + NRBF, HID), lambda i: (0, 0)),
                  pl.BlockSpec((1, HID), lambda i: (0, 0)),
                  pl.BlockSpec((HID, HID), lambda i: (0, 0)),
                  pl.BlockSpec((1, HID), lambda i: (0, 0))],
        out_specs=[pl.BlockSpec((BE, HID), lambda i: (i, 0)),
                   pl.BlockSpec((BE, HID), lambda i: (i, 0)),
                   pl.BlockSpec((BE, 16), lambda i: (i, 0))],
        out_shape=[jax.ShapeDtypeStruct((E, HID), jnp.float32),
                   jax.ShapeDtypeStruct((E, HID), jnp.float32),
                   jax.ShapeDtypeStruct((E, 16), jnp.float32)],
    )(Xs, Xd, Cs, Cd, We, be, W1, b1)


def _tc_pass2(e, Xs, Xd, W, b):
    def body(e_ref, xs_ref, xd_ref, w_ref, b_ref, e2_ref, m_ref):
        t = e_ref[...] + xs_ref[...] + xd_ref[...]
        m = _silu(_dot(t, w_ref[...]) + b_ref[...])
        m_ref[...] = m
        e2_ref[...] = e_ref[...] + m

    return pl.pallas_call(
        body,
        grid=(E // BE,),
        in_specs=[pl.BlockSpec((BE, HID), lambda i: (i, 0)),
                  pl.BlockSpec((BE, HID), lambda i: (i, 0)),
                  pl.BlockSpec((BE, HID), lambda i: (i, 0)),
                  pl.BlockSpec((HID, HID), lambda i: (0, 0)),
                  pl.BlockSpec((1, HID), lambda i: (0, 0))],
        out_specs=[pl.BlockSpec((BE, HID), lambda i: (i, 0)),
                   pl.BlockSpec((BE, HID), lambda i: (i, 0))],
        out_shape=[jax.ShapeDtypeStruct((E, HID), jnp.float32),
                   jax.ShapeDtypeStruct((E, HID), jnp.float32)],
    )(e, Xs, Xd, W, b)


def _tc_pass3(e, Xs, Xd, unitp, W, b, wox8, box8):
    def body(e_ref, xs_ref, xd_ref, u_ref, w_ref, b_ref, wox_ref, box_ref,
             m_ref, uv_ref):
        t = e_ref[...] + xs_ref[...] + xd_ref[...]
        m = _silu(_dot(t, w_ref[...]) + b_ref[...])
        e3 = e_ref[...] + m
        scal = _dot(e3, wox_ref[...]) + box_ref[...]
        m_ref[...] = m
        uv_ref[...] = scal[:, 0:1] * u_ref[...]

    return pl.pallas_call(
        body,
        grid=(E // BE,),
        in_specs=[pl.BlockSpec((BE, HID), lambda i: (i, 0)),
                  pl.BlockSpec((BE, HID), lambda i: (i, 0)),
                  pl.BlockSpec((BE, HID), lambda i: (i, 0)),
                  pl.BlockSpec((BE, 16), lambda i: (i, 0)),
                  pl.BlockSpec((HID, HID), lambda i: (0, 0)),
                  pl.BlockSpec((1, HID), lambda i: (0, 0)),
                  pl.BlockSpec((HID, 8), lambda i: (0, 0)),
                  pl.BlockSpec((1, 8), lambda i: (0, 0))],
        out_specs=[pl.BlockSpec((BE, HID), lambda i: (i, 0)),
                   pl.BlockSpec((BE, 16), lambda i: (i, 0))],
        out_shape=[jax.ShapeDtypeStruct((E, HID), jnp.float32),
                   jax.ShapeDtypeStruct((E, 16), jnp.float32)],
    )(e, Xs, Xd, unitp, W, b, wox8, box8)


def _tc_node(x, a0, a1, W, b):
    def body(x_ref, a0_ref, a1_ref, w_ref, b_ref, o_ref):
        agg = a0_ref[...] + a1_ref[...]
        o_ref[...] = x_ref[...] + _silu(_dot(agg, w_ref[...]) + b_ref[...])

    return pl.pallas_call(
        body,
        grid=(N // BN,),
        in_specs=[pl.BlockSpec((BN, HID), lambda i: (i, 0)),
                  pl.BlockSpec((BN, HID), lambda i: (i, 0)),
                  pl.BlockSpec((BN, HID), lambda i: (i, 0)),
                  pl.BlockSpec((HID, HID), lambda i: (0, 0)),
                  pl.BlockSpec((1, HID), lambda i: (0, 0))],
        out_specs=pl.BlockSpec((BN, HID), lambda i: (i, 0)),
        out_shape=jax.ShapeDtypeStruct((N, HID), jnp.float32),
    )(x, a0, a1, W, b)


def _tc_node3(x, a0, a1, u0, u1, W, b):
    def body(x_ref, a0_ref, a1_ref, u0_ref, u1_ref, w_ref, b_ref,
             x3_ref, eps_ref):
        agg = a0_ref[...] + a1_ref[...]
        x3_ref[...] = x_ref[...] + _silu(_dot(agg, w_ref[...]) + b_ref[...])
        eps_ref[...] = u0_ref[...] + u1_ref[...]

    return pl.pallas_call(
        body,
        grid=(N // BN,),
        in_specs=[pl.BlockSpec((BN, HID), lambda i: (i, 0)),
                  pl.BlockSpec((BN, HID), lambda i: (i, 0)),
                  pl.BlockSpec((BN, HID), lambda i: (i, 0)),
                  pl.BlockSpec((BN, 16), lambda i: (i, 0)),
                  pl.BlockSpec((BN, 16), lambda i: (i, 0)),
                  pl.BlockSpec((HID, HID), lambda i: (0, 0)),
                  pl.BlockSpec((1, HID), lambda i: (0, 0))],
        out_specs=[pl.BlockSpec((BN, HID), lambda i: (i, 0)),
                   pl.BlockSpec((BN, 16), lambda i: (i, 0))],
        out_shape=[jax.ShapeDtypeStruct((N, HID), jnp.float32),
                   jax.ShapeDtypeStruct((N, 16), jnp.float32)],
    )(x, a0, a1, u0, u1, W, b)


def _tc_outh(x, W, b):
    def body(x_ref, w_ref, b_ref, o_ref):
        o_ref[...] = _dot(x_ref[...], w_ref[...]) + b_ref[...]

    return pl.pallas_call(
        body,
        grid=(N // BN,),
        in_specs=[pl.BlockSpec((BN, HID), lambda i: (i, 0)),
                  pl.BlockSpec((HID, NOUT), lambda i: (0, 0)),
                  pl.BlockSpec((1, NOUT), lambda i: (0, 0))],
        out_specs=pl.BlockSpec((BN, NOUT), lambda i: (i, 0)),
        out_shape=jax.ShapeDtypeStruct((N, NOUT), jnp.float32),
    )(x, W, b)


def _lattice(lengths, angles):
    a, b, c = lengths[:, 0], lengths[:, 1], lengths[:, 2]
    al = jnp.deg2rad(angles[:, 0])
    be = jnp.deg2rad(angles[:, 1])
    ga = jnp.deg2rad(angles[:, 2])
    cos_a, cos_b, cos_g = jnp.cos(al), jnp.cos(be), jnp.cos(ga)
    sin_g = jnp.sin(ga)
    zeros = jnp.zeros_like(a)
    v1 = jnp.stack([a, zeros, zeros], -1)
    v2 = jnp.stack([b * cos_g, b * sin_g, zeros], -1)
    cx = cos_b
    cy = (cos_a - cos_b * cos_g) / sin_g
    cz = jnp.sqrt(jnp.clip(1.0 - cx ** 2 - cy ** 2, 1e-8, None))
    v3 = jnp.stack([c * cx, c * cy, c * cz], -1)
    return jnp.stack([v1, v2, v3], 1)


def kernel(z, frac_x, h, num_atoms, lengths, angles, edge_index,
           W_emb, b_emb, W_edge, b_edge, W_msg, b_msg, W_upd, b_upd,
           W_out_h, b_out_h, W_out_x, b_out_x):
    src3 = edge_index[0].reshape(NW, NCHUNK, CHUNK)
    dst3 = edge_index[1].reshape(NW, NCHUNK, CHUNK)

    zb = jnp.repeat(z, num_atoms, axis=0, total_repeat_length=N)
    lat = _lattice(lengths, angles)
    latb = jnp.repeat(lat, num_atoms, axis=0, total_repeat_length=N)
    cart = jnp.einsum('ni,nij->nj', frac_x, latb)
    cart16 = jnp.pad(cart, ((0, 0), (0, 13)))

    x0 = _tc_embed(h, zb, W_emb[:256], W_emb[256:], b_emb.reshape(1, -1))

    Xs1, Xd1 = _sc_gather2(x0, src3, dst3)
    Cs, Cd = _sc_gather2(cart16, src3, dst3, tiled=False)
    e1, m1, unitp = _tc_pass1(Xs1, Xd1, Cs, Cd, W_edge, b_edge.reshape(1, -1),
                              W_msg[0], b_msg[0].reshape(1, -1))
    P1 = _sc_scatter(m1, dst3)
    x1 = _tc_node(x0, P1[0], P1[1], W_upd[0], b_upd[0].reshape(1, -1))

    Xs2, Xd2 = _sc_gather2(x1, src3, dst3)
    e2, m2 = _tc_pass2(e1, Xs2, Xd2, W_msg[1], b_msg[1].reshape(1, -1))
    P2 = _sc_scatter(m2, dst3)
    x2 = _tc_node(x1, P2[0], P2[1], W_upd[1], b_upd[1].reshape(1, -1))

    wox8 = jnp.pad(W_out_x, ((0, 0), (0, 7)))
    box8 = jnp.pad(b_out_x, (0, 7)).reshape(1, 8)
    Xs3, Xd3 = _sc_gather2(x2, src3, dst3)
    m3, uv = _tc_pass3(e2, Xs3, Xd3, unitp, W_msg[2], b_msg[2].reshape(1, -1),
                       wox8, box8)
    P3 = _sc_scatter(m3, dst3)
    U = _sc_scatter(uv, dst3, tiled=False)
    x3, epsp = _tc_node3(x2, P3[0], P3[1], U[0], U[1],
                         W_upd[2], b_upd[2].reshape(1, -1))

    pred_eps_h = _tc_outh(x3, W_out_h, b_out_h.reshape(1, -1))
    pred_eps_x = epsp[:, :3]
    return (pred_eps_x, pred_eps_h)


# speedup vs baseline: 4.1911x; 1.0598x over previous
import functools

import jax
import jax.numpy as jnp
from jax import lax
from jax.experimental import pallas as pl
from jax.experimental.pallas import tpu as pltpu
from jax.experimental.pallas import tpu_sc as plsc

N = 10000
E = 320000
HID = 128
NRBF = 16
NOUT = 100
CUTOFF = 6.0
WX = HID + 16

NC = 2
NS = 16
NW = NC * NS
CHUNK = 40
K = 5
EH = E // 2


def _silu(v):
    return v * jax.nn.sigmoid(v)


def _dot(a, b):
    return jnp.dot(a, b, preferred_element_type=jnp.float32)


def _sc_gather2(table, src2, dst2, tiled=True):
    W = table.shape[1]
    PER = src2.shape[1]
    NE = NW * PER
    NGROUP = PER // (K * CHUNK)
    mesh = plsc.VectorSubcoreMesh(core_axis_name="c", subcore_axis_name="s")
    params = (pltpu.CompilerParams() if tiled
              else pltpu.CompilerParams(use_tc_tiling_on_sc=False))

    @functools.partial(
        pl.kernel,
        out_type=[jax.ShapeDtypeStruct((NE, W), jnp.float32),
                  jax.ShapeDtypeStruct((NE, W), jnp.float32)],
        mesh=mesh,
        scratch_types=([pltpu.VMEM((PER,), jnp.int32)]
                       + [pltpu.VMEM((CHUNK, W), jnp.float32)
                          for _ in range(K)]
                       + [pltpu.SemaphoreType.DMA, pltpu.SemaphoreType.DMA,
                          pltpu.VMEM_SHARED((N, W), jnp.float32)]),
        compiler_params=params,
    )
    def k(tbl, s_idx, d_idx, o1, o2, idx1, r0, r1, r2, r3, r4,
          gsem, ssem, shtbl):
        cid = lax.axis_index("c")
        sid = lax.axis_index("s")
        wid = sid * NC + cid
        bufs = [r0, r1, r2, r3, r4]

        @pl.when(sid == 0)
        def _stage():
            pltpu.sync_copy(tbl, shtbl)
        plsc.subcore_barrier()

        def run(idx2_hbm, out_hbm):
            pltpu.sync_copy(idx2_hbm.at[wid], idx1)

            def group(g, carry):
                descs = [pltpu.async_copy(
                    shtbl.at[idx1.at[pl.ds((g * K + b) * CHUNK, CHUNK)]],
                    bufs[b], gsem)
                    for b in range(K)]
                for d in descs:
                    d.wait()
                base = wid * PER + g * (K * CHUNK)
                descs = [pltpu.async_copy(
                    bufs[b], out_hbm.at[pl.ds(base + b * CHUNK, CHUNK)], ssem)
                    for b in range(K)]
                for d in descs:
                    d.wait()
                return carry
            lax.fori_loop(0, NGROUP, group, 0)

        run(s_idx, o1)
        run(d_idx, o2)

    return k(table, src2, dst2)


def _sc_scatter(rows_in, dst3, tiled=True):
    W = rows_in.shape[1]
    PER = dst3.shape[1] * dst3.shape[2]
    NGROUP = PER // (K * CHUNK)
    mesh = plsc.VectorSubcoreMesh(core_axis_name="c", subcore_axis_name="s")
    zer = jnp.zeros((N, W), jnp.float32)
    params = (pltpu.CompilerParams() if tiled
              else pltpu.CompilerParams(use_tc_tiling_on_sc=False))

    @functools.partial(
        pl.kernel,
        out_type=jax.ShapeDtypeStruct((NC, N, W), jnp.float32),
        mesh=mesh,
        scratch_types=([pltpu.VMEM((K, CHUNK), jnp.int32)]
                       + [pltpu.VMEM((CHUNK, W), jnp.float32)
                          for _ in range(K)]
                       + [pltpu.SemaphoreType.DMA, pltpu.SemaphoreType.DMA,
                          pltpu.SemaphoreType.DMA,
                          pltpu.VMEM_SHARED((N, W), jnp.float32)]),
        compiler_params=params,
    )
    def k(rows_hbm, d_idx, zer_hbm, out, idxb, r0, r1, r2, r3, r4,
          isem, lsem, asem, agg):
        cid = lax.axis_index("c")
        sid = lax.axis_index("s")
        wid = sid * NC + cid
        bufs = [r0, r1, r2, r3, r4]

        @pl.when(sid == 0)
        def _zero():
            pltpu.sync_copy(zer_hbm, agg)
        plsc.subcore_barrier()

        def group(g, carry):
            base = wid * PER + g * (K * CHUNK)
            idescs = [pltpu.async_copy(
                d_idx.at[wid, g * K + b], idxb.at[b], isem)
                for b in range(K)]
            ldescs = [pltpu.async_copy(
                rows_hbm.at[pl.ds(base + b * CHUNK, CHUNK)], bufs[b], lsem)
                for b in range(K)]
            for d in idescs:
                d.wait()
            for d in ldescs:
                d.wait()
            descs = [pltpu.async_copy(
                bufs[b], agg.at[idxb.at[b]], asem, add=True)
                for b in range(K)]
            for d in descs:
                d.wait()
            return carry
        lax.fori_loop(0, NGROUP, group, 0)

        plsc.subcore_barrier()

        @pl.when(sid == 0)
        def _dump():
            pltpu.sync_copy(agg, out.at[cid])

    return k(rows_in, dst3, zer)


BN = 2000
BE = 2000


def _tc_embed(h, zb, W_h, W_z, b):
    def body(h_ref, z_ref, wh_ref, wz_ref, b_ref, o_ref):
        acc = _dot(h_ref[...], wh_ref[...]) + _dot(z_ref[...], wz_ref[...]) + b_ref[...]
        o_ref[...] = _silu(acc)

    return pl.pallas_call(
        body,
        grid=(N // BN,),
        in_specs=[pl.BlockSpec((BN, 256), lambda i: (i, 0)),
                  pl.BlockSpec((BN, 256), lambda i: (i, 0)),
                  pl.BlockSpec((256, HID), lambda i: (0, 0)),
                  pl.BlockSpec((256, HID), lambda i: (0, 0)),
                  pl.BlockSpec((1, HID), lambda i: (0, 0))],
        out_specs=pl.BlockSpec((BN, HID), lambda i: (i, 0)),
        out_shape=jax.ShapeDtypeStruct((N, HID), jnp.float32),
    )(h, zb, W_h, W_z, b)


def _tc_pass1(Xs, Xd, Cs, Cd, hoff, We, be, W1, b1):
    ne = Xs.shape[0]
    def body(xs_ref, xd_ref, cs_ref, cd_ref, we_ref, be_ref, w1_ref, b1_ref,
             e1_ref, m1_ref, u_ref):
        xs = xs_ref[...]
        xd = xd_ref[...]
        dv = cd_ref[...] - cs_ref[...]
        dist = jnp.sqrt(jnp.sum(dv * dv, axis=1, keepdims=True) + 1e-9)
        centers = (CUTOFF / (NRBF - 1)) * lax.broadcasted_iota(
            jnp.int32, (1, NRBF), 1).astype(jnp.float32)
        diff = dist - centers
        rbf = jnp.exp(-10.0 * diff * diff)
        pre = (_dot(xs, we_ref[:HID]) + _dot(xd, we_ref[HID:2 * HID])
               + _dot(rbf, we_ref[2 * HID:]) + be_ref[...])
        e0 = _silu(pre)
        t = e0 + xs + xd
        m1 = _silu(_dot(t, w1_ref[...]) + b1_ref[...])
        e1_ref[...] = e0 + m1
        m1_ref[...] = m1
        u_ref[...] = dv / dist

    return pl.pallas_call(
        body,
        grid=(ne // BE,),
        in_specs=[pl.BlockSpec((BE, HID), lambda i: (i, 0)),
                  pl.BlockSpec((BE, HID), lambda i: (i, 0)),
                  pl.BlockSpec((BE, 16), lambda i: (i + hoff, 0)),
                  pl.BlockSpec((BE, 16), lambda i: (i + hoff, 0)),
                  pl.BlockSpec((2 * HID + NRBF, HID), lambda i: (0, 0)),
                  pl.BlockSpec((1, HID), lambda i: (0, 0)),
                  pl.BlockSpec((HID, HID), lambda i: (0, 0)),
                  pl.BlockSpec((1, HID), lambda i: (0, 0))],
        out_specs=[pl.BlockSpec((BE, HID), lambda i: (i, 0)),
                   pl.BlockSpec((BE, HID), lambda i: (i, 0)),
                   pl.BlockSpec((BE, 16), lambda i: (i, 0))],
        out_shape=[jax.ShapeDtypeStruct((ne, HID), jnp.float32),
                   jax.ShapeDtypeStruct((ne, HID), jnp.float32),
                   jax.ShapeDtypeStruct((ne, 16), jnp.float32)],
    )(Xs, Xd, Cs, Cd, We, be, W1, b1)


def _tc_pass2(e, Xs, Xd, W, b):
    ne = Xs.shape[0]

    def body(e_ref, xs_ref, xd_ref, w_ref, b_ref, e2_ref, m_ref):
        t = e_ref[...] + xs_ref[...] + xd_ref[...]
        m = _silu(_dot(t, w_ref[...]) + b_ref[...])
        m_ref[...] = m
        e2_ref[...] = e_ref[...] + m

    return pl.pallas_call(
        body,
        grid=(ne // BE,),
        in_specs=[pl.BlockSpec((BE, HID), lambda i: (i, 0)),
                  pl.BlockSpec((BE, HID), lambda i: (i, 0)),
                  pl.BlockSpec((BE, HID), lambda i: (i, 0)),
                  pl.BlockSpec((HID, HID), lambda i: (0, 0)),
                  pl.BlockSpec((1, HID), lambda i: (0, 0))],
        out_specs=[pl.BlockSpec((BE, HID), lambda i: (i, 0)),
                   pl.BlockSpec((BE, HID), lambda i: (i, 0))],
        out_shape=[jax.ShapeDtypeStruct((ne, HID), jnp.float32),
                   jax.ShapeDtypeStruct((ne, HID), jnp.float32)],
    )(e, Xs, Xd, W, b)


def _tc_pass3(e, Xs, Xd, unitp, W, b, wox8, box8):
    ne = Xs.shape[0]

    def body(e_ref, xs_ref, xd_ref, u_ref, w_ref, b_ref, wox_ref, box_ref,
             m_ref, uv_ref):
        t = e_ref[...] + xs_ref[...] + xd_ref[...]
        m = _silu(_dot(t, w_ref[...]) + b_ref[...])
        e3 = e_ref[...] + m
        scal = _dot(e3, wox_ref[...]) + box_ref[...]
        m_ref[...] = m
        uv_ref[...] = scal[:, 0:1] * u_ref[...]

    return pl.pallas_call(
        body,
        grid=(ne // BE,),
        in_specs=[pl.BlockSpec((BE, HID), lambda i: (i, 0)),
                  pl.BlockSpec((BE, HID), lambda i: (i, 0)),
                  pl.BlockSpec((BE, HID), lambda i: (i, 0)),
                  pl.BlockSpec((BE, 16), lambda i: (i, 0)),
                  pl.BlockSpec((HID, HID), lambda i: (0, 0)),
                  pl.BlockSpec((1, HID), lambda i: (0, 0)),
                  pl.BlockSpec((HID, 8), lambda i: (0, 0)),
                  pl.BlockSpec((1, 8), lambda i: (0, 0))],
        out_specs=[pl.BlockSpec((BE, HID), lambda i: (i, 0)),
                   pl.BlockSpec((BE, 16), lambda i: (i, 0))],
        out_shape=[jax.ShapeDtypeStruct((ne, HID), jnp.float32),
                   jax.ShapeDtypeStruct((ne, 16), jnp.float32)],
    )(e, Xs, Xd, unitp, W, b, wox8, box8)


def _tc_node(x, aggs, W, b):
    def body(x_ref, a0_ref, a1_ref, a2_ref, a3_ref, w_ref, b_ref, o_ref):
        agg = (a0_ref[...] + a1_ref[...]) + (a2_ref[...] + a3_ref[...])
        o_ref[...] = x_ref[...] + _silu(_dot(agg, w_ref[...]) + b_ref[...])

    return pl.pallas_call(
        body,
        grid=(N // BN,),
        in_specs=[pl.BlockSpec((BN, HID), lambda i: (i, 0))]
                 + [pl.BlockSpec((BN, HID), lambda i: (i, 0))] * 4
                 + [pl.BlockSpec((HID, HID), lambda i: (0, 0)),
                    pl.BlockSpec((1, HID), lambda i: (0, 0))],
        out_specs=pl.BlockSpec((BN, HID), lambda i: (i, 0)),
        out_shape=jax.ShapeDtypeStruct((N, HID), jnp.float32),
    )(x, *aggs, W, b)


def _tc_node3(x, aggs, us, W, b):
    def body(x_ref, a0_ref, a1_ref, a2_ref, a3_ref,
             u0_ref, u1_ref, u2_ref, u3_ref, w_ref, b_ref,
             x3_ref, eps_ref):
        agg = (a0_ref[...] + a1_ref[...]) + (a2_ref[...] + a3_ref[...])
        x3_ref[...] = x_ref[...] + _silu(_dot(agg, w_ref[...]) + b_ref[...])
        eps_ref[...] = ((u0_ref[...] + u1_ref[...])
                        + (u2_ref[...] + u3_ref[...]))

    return pl.pallas_call(
        body,
        grid=(N // BN,),
        in_specs=[pl.BlockSpec((BN, HID), lambda i: (i, 0))]
                 + [pl.BlockSpec((BN, HID), lambda i: (i, 0))] * 4
                 + [pl.BlockSpec((BN, 16), lambda i: (i, 0))] * 4
                 + [pl.BlockSpec((HID, HID), lambda i: (0, 0)),
                    pl.BlockSpec((1, HID), lambda i: (0, 0))],
        out_specs=[pl.BlockSpec((BN, HID), lambda i: (i, 0)),
                   pl.BlockSpec((BN, 16), lambda i: (i, 0))],
        out_shape=[jax.ShapeDtypeStruct((N, HID), jnp.float32),
                   jax.ShapeDtypeStruct((N, 16), jnp.float32)],
    )(x, *aggs, *us, W, b)


def _tc_outh(x, W, b):
    def body(x_ref, w_ref, b_ref, o_ref):
        o_ref[...] = _dot(x_ref[...], w_ref[...]) + b_ref[...]

    return pl.pallas_call(
        body,
        grid=(N // BN,),
        in_specs=[pl.BlockSpec((BN, HID), lambda i: (i, 0)),
                  pl.BlockSpec((HID, NOUT), lambda i: (0, 0)),
                  pl.BlockSpec((1, NOUT), lambda i: (0, 0))],
        out_specs=pl.BlockSpec((BN, NOUT), lambda i: (i, 0)),
        out_shape=jax.ShapeDtypeStruct((N, NOUT), jnp.float32),
    )(x, W, b)


def _lattice(lengths, angles):
    a, b, c = lengths[:, 0], lengths[:, 1], lengths[:, 2]
    al = jnp.deg2rad(angles[:, 0])
    be = jnp.deg2rad(angles[:, 1])
    ga = jnp.deg2rad(angles[:, 2])
    cos_a, cos_b, cos_g = jnp.cos(al), jnp.cos(be), jnp.cos(ga)
    sin_g = jnp.sin(ga)
    zeros = jnp.zeros_like(a)
    v1 = jnp.stack([a, zeros, zeros], -1)
    v2 = jnp.stack([b * cos_g, b * sin_g, zeros], -1)
    cx = cos_b
    cy = (cos_a - cos_b * cos_g) / sin_g
    cz = jnp.sqrt(jnp.clip(1.0 - cx ** 2 - cy ** 2, 1e-8, None))
    v3 = jnp.stack([c * cx, c * cy, c * cz], -1)
    return jnp.stack([v1, v2, v3], 1)


def kernel(z, frac_x, h, num_atoms, lengths, angles, edge_index,
           W_emb, b_emb, W_edge, b_edge, W_msg, b_msg, W_upd, b_upd,
           W_out_h, b_out_h, W_out_x, b_out_x):
    perh = EH // NW
    srcs, dsts, dst3s = [], [], []
    for hh in range(2):
        sl = slice(hh * EH, (hh + 1) * EH)
        srcs.append(edge_index[0, sl].reshape(NW, perh))
        dsts.append(edge_index[1, sl].reshape(NW, perh))
        dst3s.append(edge_index[1, sl].reshape(NW, perh // CHUNK, CHUNK))
    src_full = edge_index[0].reshape(NW, E // NW)
    dst_full = edge_index[1].reshape(NW, E // NW)

    zb = jnp.repeat(z, num_atoms, axis=0, total_repeat_length=N)
    lat = _lattice(lengths, angles)
    latb = jnp.repeat(lat, num_atoms, axis=0, total_repeat_length=N)
    cart = jnp.einsum('ni,nij->nj', frac_x, latb)
    cart16 = jnp.pad(cart, ((0, 0), (0, 13)))

    x0 = _tc_embed(h, zb, W_emb[:256], W_emb[256:], b_emb.reshape(1, -1))
    hblk = EH // BE

    Cs, Cd = _sc_gather2(cart16, src_full, dst_full, tiled=False)

    es, ms, us = [], [], []
    for hh in range(2):
        Xs, Xd = _sc_gather2(x0, srcs[hh], dsts[hh])
        e1h, m1h, uh = _tc_pass1(Xs, Xd, Cs, Cd, hh * hblk,
                                 W_edge, b_edge.reshape(1, -1),
                                 W_msg[0], b_msg[0].reshape(1, -1))
        es.append(e1h)
        ms.append(m1h)
        us.append(uh)
    P1 = [_sc_scatter(ms[hh], dst3s[hh]) for hh in range(2)]
    x1 = _tc_node(x0, [P1[0][0], P1[0][1], P1[1][0], P1[1][1]],
                  W_upd[0], b_upd[0].reshape(1, -1))

    e2s, m2s = [], []
    for hh in range(2):
        Xs, Xd = _sc_gather2(x1, srcs[hh], dsts[hh])
        e2h, m2h = _tc_pass2(es[hh], Xs, Xd, W_msg[1], b_msg[1].reshape(1, -1))
        e2s.append(e2h)
        m2s.append(m2h)
    P2 = [_sc_scatter(m2s[hh], dst3s[hh]) for hh in range(2)]
    x2 = _tc_node(x1, [P2[0][0], P2[0][1], P2[1][0], P2[1][1]],
                  W_upd[1], b_upd[1].reshape(1, -1))

    wox8 = jnp.pad(W_out_x, ((0, 0), (0, 7)))
    box8 = jnp.pad(b_out_x, (0, 7)).reshape(1, 8)
    m3s, uvs = [], []
    for hh in range(2):
        Xs, Xd = _sc_gather2(x2, srcs[hh], dsts[hh])
        m3h, uvh = _tc_pass3(e2s[hh], Xs, Xd, us[hh],
                             W_msg[2], b_msg[2].reshape(1, -1), wox8, box8)
        m3s.append(m3h)
        uvs.append(uvh)
    P3 = [_sc_scatter(m3s[hh], dst3s[hh]) for hh in range(2)]
    U = [_sc_scatter(uvs[hh], dst3s[hh], tiled=False) for hh in range(2)]
    x3, epsp = _tc_node3(x2, [P3[0][0], P3[0][1], P3[1][0], P3[1][1]],
                         [U[0][0], U[0][1], U[1][0], U[1][1]],
                         W_upd[2], b_upd[2].reshape(1, -1))

    pred_eps_h = _tc_outh(x3, W_out_h, b_out_h.reshape(1, -1))
    pred_eps_x = epsp[:, :3]
    return (pred_eps_x, pred_eps_h)


# speedup vs baseline: 4.3195x; 1.0306x over previous
import functools

import jax
import jax.numpy as jnp
from jax import lax
from jax.experimental import pallas as pl
from jax.experimental.pallas import tpu as pltpu
from jax.experimental.pallas import tpu_sc as plsc

N = 10000
E = 320000
HID = 128
NRBF = 16
NOUT = 100
CUTOFF = 6.0
WX = HID + 16

NC = 2
NS = 16
NW = NC * NS
CHUNK = 40
K = 5
EH = E // 2


def _silu(v):
    return v * jax.nn.sigmoid(v)


def _dot(a, b):
    return jnp.dot(a, b, preferred_element_type=jnp.float32)


def _sc_gather2(table, src2, dst2, tiled=True):
    W = table.shape[1]
    PER = src2.shape[1]
    NE = NW * PER
    NGROUP = PER // (K * CHUNK)
    mesh = plsc.VectorSubcoreMesh(core_axis_name="c", subcore_axis_name="s")
    params = (pltpu.CompilerParams() if tiled
              else pltpu.CompilerParams(use_tc_tiling_on_sc=False))

    @functools.partial(
        pl.kernel,
        out_type=[jax.ShapeDtypeStruct((NE, W), jnp.float32),
                  jax.ShapeDtypeStruct((NE, W), jnp.float32)],
        mesh=mesh,
        scratch_types=([pltpu.VMEM((PER,), jnp.int32)]
                       + [pltpu.VMEM((CHUNK, W), jnp.float32)
                          for _ in range(K)]
                       + [pltpu.SemaphoreType.DMA, pltpu.SemaphoreType.DMA,
                          pltpu.VMEM_SHARED((N, W), jnp.float32)]),
        compiler_params=params,
    )
    def k(tbl, s_idx, d_idx, o1, o2, idx1, r0, r1, r2, r3, r4,
          gsem, ssem, shtbl):
        cid = lax.axis_index("c")
        sid = lax.axis_index("s")
        wid = sid * NC + cid
        bufs = [r0, r1, r2, r3, r4]

        @pl.when(sid == 0)
        def _stage():
            pltpu.sync_copy(tbl, shtbl)
        plsc.subcore_barrier()

        def run(idx2_hbm, out_hbm):
            pltpu.sync_copy(idx2_hbm.at[wid], idx1)

            def group(g, carry):
                descs = [pltpu.async_copy(
                    shtbl.at[idx1.at[pl.ds((g * K + b) * CHUNK, CHUNK)]],
                    bufs[b], gsem)
                    for b in range(K)]
                for d in descs:
                    d.wait()
                base = wid * PER + g * (K * CHUNK)
                descs = [pltpu.async_copy(
                    bufs[b], out_hbm.at[pl.ds(base + b * CHUNK, CHUNK)], ssem)
                    for b in range(K)]
                for d in descs:
                    d.wait()
                return carry
            lax.fori_loop(0, NGROUP, group, 0)

        run(s_idx, o1)
        run(d_idx, o2)

    return k(table, src2, dst2)


def _sc_scatter(rows_in, dst3, tiled=True):
    W = rows_in.shape[1]
    PER = dst3.shape[1] * dst3.shape[2]
    NGROUP = PER // (K * CHUNK)
    mesh = plsc.VectorSubcoreMesh(core_axis_name="c", subcore_axis_name="s")
    zer = jnp.zeros((N, W), jnp.float32)
    params = (pltpu.CompilerParams() if tiled
              else pltpu.CompilerParams(use_tc_tiling_on_sc=False))

    @functools.partial(
        pl.kernel,
        out_type=jax.ShapeDtypeStruct((NC, N, W), jnp.float32),
        mesh=mesh,
        scratch_types=([pltpu.VMEM((K, CHUNK), jnp.int32)]
                       + [pltpu.VMEM((CHUNK, W), jnp.float32)
                          for _ in range(K)]
                       + [pltpu.SemaphoreType.DMA, pltpu.SemaphoreType.DMA,
                          pltpu.SemaphoreType.DMA,
                          pltpu.VMEM_SHARED((N, W), jnp.float32)]),
        compiler_params=params,
    )
    def k(rows_hbm, d_idx, zer_hbm, out, idxb, r0, r1, r2, r3, r4,
          isem, lsem, asem, agg):
        cid = lax.axis_index("c")
        sid = lax.axis_index("s")
        wid = sid * NC + cid
        bufs = [r0, r1, r2, r3, r4]

        @pl.when(sid == 0)
        def _zero():
            pltpu.sync_copy(zer_hbm, agg)
        plsc.subcore_barrier()

        def group(g, carry):
            base = wid * PER + g * (K * CHUNK)
            idescs = [pltpu.async_copy(
                d_idx.at[wid, g * K + b], idxb.at[b], isem)
                for b in range(K)]
            ldescs = [pltpu.async_copy(
                rows_hbm.at[pl.ds(base + b * CHUNK, CHUNK)], bufs[b], lsem)
                for b in range(K)]
            for d in idescs:
                d.wait()
            for d in ldescs:
                d.wait()
            descs = [pltpu.async_copy(
                bufs[b], agg.at[idxb.at[b]], asem, add=True)
                for b in range(K)]
            for d in descs:
                d.wait()
            return carry
        lax.fori_loop(0, NGROUP, group, 0)

        plsc.subcore_barrier()

        @pl.when(sid == 0)
        def _dump():
            pltpu.sync_copy(agg, out.at[cid])

    return k(rows_in, dst3, zer)


BN = 2000
BE = 2000


def _tc_embed(h, zb, W_h, W_z, b):
    def body(h_ref, z_ref, wh_ref, wz_ref, b_ref, o_ref):
        acc = _dot(h_ref[...], wh_ref[...]) + _dot(z_ref[...], wz_ref[...]) + b_ref[...]
        o_ref[...] = _silu(acc)

    return pl.pallas_call(
        body,
        grid=(N // BN,),
        in_specs=[pl.BlockSpec((BN, 256), lambda i: (i, 0)),
                  pl.BlockSpec((BN, 256), lambda i: (i, 0)),
                  pl.BlockSpec((256, HID), lambda i: (0, 0)),
                  pl.BlockSpec((256, HID), lambda i: (0, 0)),
                  pl.BlockSpec((1, HID), lambda i: (0, 0))],
        out_specs=pl.BlockSpec((BN, HID), lambda i: (i, 0)),
        out_shape=jax.ShapeDtypeStruct((N, HID), jnp.float32),
    )(h, zb, W_h, W_z, b)


def _tc_pass1(Xs, Xd, Cs, Cd, hoff, We, be, W1, b1):
    ne = Xs.shape[0]
    def body(xs_ref, xd_ref, cs_ref, cd_ref, we_ref, be_ref, w1_ref, b1_ref,
             e1_ref, m1_ref):
        xs = xs_ref[...]
        xd = xd_ref[...]
        dv = cd_ref[...] - cs_ref[...]
        dist = jnp.sqrt(jnp.sum(dv * dv, axis=1, keepdims=True) + 1e-9)
        centers = (CUTOFF / (NRBF - 1)) * lax.broadcasted_iota(
            jnp.int32, (1, NRBF), 1).astype(jnp.float32)
        diff = dist - centers
        rbf = jnp.exp(-10.0 * diff * diff)
        pre = (_dot(xs, we_ref[:HID]) + _dot(xd, we_ref[HID:2 * HID])
               + _dot(rbf, we_ref[2 * HID:]) + be_ref[...])
        e0 = _silu(pre)
        t = e0 + xs + xd
        m1 = _silu(_dot(t, w1_ref[...]) + b1_ref[...])
        e1_ref[...] = (e0 + m1).astype(jnp.bfloat16)
        m1_ref[...] = m1

    return pl.pallas_call(
        body,
        grid=(ne // BE,),
        in_specs=[pl.BlockSpec((BE, HID), lambda i: (i, 0)),
                  pl.BlockSpec((BE, HID), lambda i: (i, 0)),
                  pl.BlockSpec((BE, 16), lambda i: (i + hoff, 0)),
                  pl.BlockSpec((BE, 16), lambda i: (i + hoff, 0)),
                  pl.BlockSpec((2 * HID + NRBF, HID), lambda i: (0, 0)),
                  pl.BlockSpec((1, HID), lambda i: (0, 0)),
                  pl.BlockSpec((HID, HID), lambda i: (0, 0)),
                  pl.BlockSpec((1, HID), lambda i: (0, 0))],
        out_specs=[pl.BlockSpec((BE, HID), lambda i: (i, 0)),
                   pl.BlockSpec((BE, HID), lambda i: (i, 0))],
        out_shape=[jax.ShapeDtypeStruct((ne, HID), jnp.bfloat16),
                   jax.ShapeDtypeStruct((ne, HID), jnp.float32)],
    )(Xs, Xd, Cs, Cd, We, be, W1, b1)


def _tc_pass2(e, Xs, Xd, W, b):
    ne = Xs.shape[0]

    def body(e_ref, xs_ref, xd_ref, w_ref, b_ref, e2_ref, m_ref):
        ef = e_ref[...].astype(jnp.float32)
        t = ef + xs_ref[...] + xd_ref[...]
        m = _silu(_dot(t, w_ref[...]) + b_ref[...])
        m_ref[...] = m
        e2_ref[...] = (ef + m).astype(jnp.bfloat16)

    return pl.pallas_call(
        body,
        grid=(ne // BE,),
        in_specs=[pl.BlockSpec((BE, HID), lambda i: (i, 0)),
                  pl.BlockSpec((BE, HID), lambda i: (i, 0)),
                  pl.BlockSpec((BE, HID), lambda i: (i, 0)),
                  pl.BlockSpec((HID, HID), lambda i: (0, 0)),
                  pl.BlockSpec((1, HID), lambda i: (0, 0))],
        out_specs=[pl.BlockSpec((BE, HID), lambda i: (i, 0)),
                   pl.BlockSpec((BE, HID), lambda i: (i, 0))],
        out_shape=[jax.ShapeDtypeStruct((ne, HID), jnp.bfloat16),
                   jax.ShapeDtypeStruct((ne, HID), jnp.float32)],
    )(e, Xs, Xd, W, b)


def _tc_pass3(e, Xs, Xd, Cs, Cd, hoff, W, b, wox8, box8):
    ne = Xs.shape[0]

    def body(e_ref, xs_ref, xd_ref, cs_ref, cd_ref, w_ref, b_ref,
             wox_ref, box_ref, m_ref, uv_ref):
        ef = e_ref[...].astype(jnp.float32)
        t = ef + xs_ref[...] + xd_ref[...]
        m = _silu(_dot(t, w_ref[...]) + b_ref[...])
        e3 = ef + m
        scal = _dot(e3, wox_ref[...]) + box_ref[...]
        m_ref[...] = m
        dv = cd_ref[...] - cs_ref[...]
        dist = jnp.sqrt(jnp.sum(dv * dv, axis=1, keepdims=True) + 1e-9)
        uv_ref[...] = (scal[:, 0:1] / dist) * dv

    return pl.pallas_call(
        body,
        grid=(ne // BE,),
        in_specs=[pl.BlockSpec((BE, HID), lambda i: (i, 0)),
                  pl.BlockSpec((BE, HID), lambda i: (i, 0)),
                  pl.BlockSpec((BE, HID), lambda i: (i, 0)),
                  pl.BlockSpec((BE, 16), lambda i: (i + hoff, 0)),
                  pl.BlockSpec((BE, 16), lambda i: (i + hoff, 0)),
                  pl.BlockSpec((HID, HID), lambda i: (0, 0)),
                  pl.BlockSpec((1, HID), lambda i: (0, 0)),
                  pl.BlockSpec((HID, 8), lambda i: (0, 0)),
                  pl.BlockSpec((1, 8), lambda i: (0, 0))],
        out_specs=[pl.BlockSpec((BE, HID), lambda i: (i, 0)),
                   pl.BlockSpec((BE, 16), lambda i: (i, 0))],
        out_shape=[jax.ShapeDtypeStruct((ne, HID), jnp.float32),
                   jax.ShapeDtypeStruct((ne, 16), jnp.float32)],
    )(e, Xs, Xd, Cs, Cd, W, b, wox8, box8)


def _tc_node(x, aggs, W, b):
    def body(x_ref, a0_ref, a1_ref, a2_ref, a3_ref, w_ref, b_ref, o_ref):
        agg = (a0_ref[...] + a1_ref[...]) + (a2_ref[...] + a3_ref[...])
        o_ref[...] = x_ref[...] + _silu(_dot(agg, w_ref[...]) + b_ref[...])

    return pl.pallas_call(
        body,
        grid=(N // BN,),
        in_specs=[pl.BlockSpec((BN, HID), lambda i: (i, 0))]
                 + [pl.BlockSpec((BN, HID), lambda i: (i, 0))] * 4
                 + [pl.BlockSpec((HID, HID), lambda i: (0, 0)),
                    pl.BlockSpec((1, HID), lambda i: (0, 0))],
        out_specs=pl.BlockSpec((BN, HID), lambda i: (i, 0)),
        out_shape=jax.ShapeDtypeStruct((N, HID), jnp.float32),
    )(x, *aggs, W, b)


def _tc_node3(x, aggs, us, W, b):
    def body(x_ref, a0_ref, a1_ref, a2_ref, a3_ref,
             u0_ref, u1_ref, u2_ref, u3_ref, w_ref, b_ref,
             x3_ref, eps_ref):
        agg = (a0_ref[...] + a1_ref[...]) + (a2_ref[...] + a3_ref[...])
        x3_ref[...] = x_ref[...] + _silu(_dot(agg, w_ref[...]) + b_ref[...])
        eps_ref[...] = ((u0_ref[...] + u1_ref[...])
                        + (u2_ref[...] + u3_ref[...]))

    return pl.pallas_call(
        body,
        grid=(N // BN,),
        in_specs=[pl.BlockSpec((BN, HID), lambda i: (i, 0))]
                 + [pl.BlockSpec((BN, HID), lambda i: (i, 0))] * 4
                 + [pl.BlockSpec((BN, 16), lambda i: (i, 0))] * 4
                 + [pl.BlockSpec((HID, HID), lambda i: (0, 0)),
                    pl.BlockSpec((1, HID), lambda i: (0, 0))],
        out_specs=[pl.BlockSpec((BN, HID), lambda i: (i, 0)),
                   pl.BlockSpec((BN, 16), lambda i: (i, 0))],
        out_shape=[jax.ShapeDtypeStruct((N, HID), jnp.float32),
                   jax.ShapeDtypeStruct((N, 16), jnp.float32)],
    )(x, *aggs, *us, W, b)


def _tc_outh(x, W, b):
    def body(x_ref, w_ref, b_ref, o_ref):
        o_ref[...] = _dot(x_ref[...], w_ref[...]) + b_ref[...]

    return pl.pallas_call(
        body,
        grid=(N // BN,),
        in_specs=[pl.BlockSpec((BN, HID), lambda i: (i, 0)),
                  pl.BlockSpec((HID, NOUT), lambda i: (0, 0)),
                  pl.BlockSpec((1, NOUT), lambda i: (0, 0))],
        out_specs=pl.BlockSpec((BN, NOUT), lambda i: (i, 0)),
        out_shape=jax.ShapeDtypeStruct((N, NOUT), jnp.float32),
    )(x, W, b)


def _lattice(lengths, angles):
    a, b, c = lengths[:, 0], lengths[:, 1], lengths[:, 2]
    al = jnp.deg2rad(angles[:, 0])
    be = jnp.deg2rad(angles[:, 1])
    ga = jnp.deg2rad(angles[:, 2])
    cos_a, cos_b, cos_g = jnp.cos(al), jnp.cos(be), jnp.cos(ga)
    sin_g = jnp.sin(ga)
    zeros = jnp.zeros_like(a)
    v1 = jnp.stack([a, zeros, zeros], -1)
    v2 = jnp.stack([b * cos_g, b * sin_g, zeros], -1)
    cx = cos_b
    cy = (cos_a - cos_b * cos_g) / sin_g
    cz = jnp.sqrt(jnp.clip(1.0 - cx ** 2 - cy ** 2, 1e-8, None))
    v3 = jnp.stack([c * cx, c * cy, c * cz], -1)
    return jnp.stack([v1, v2, v3], 1)


def kernel(z, frac_x, h, num_atoms, lengths, angles, edge_index,
           W_emb, b_emb, W_edge, b_edge, W_msg, b_msg, W_upd, b_upd,
           W_out_h, b_out_h, W_out_x, b_out_x):
    perh = EH // NW
    srcs, dsts, dst3s = [], [], []
    for hh in range(2):
        sl = slice(hh * EH, (hh + 1) * EH)
        srcs.append(edge_index[0, sl].reshape(NW, perh))
        dsts.append(edge_index[1, sl].reshape(NW, perh))
        dst3s.append(edge_index[1, sl].reshape(NW, perh // CHUNK, CHUNK))
    src_full = edge_index[0].reshape(NW, E // NW)
    dst_full = edge_index[1].reshape(NW, E // NW)

    zb = jnp.repeat(z, num_atoms, axis=0, total_repeat_length=N)
    lat = _lattice(lengths, angles)
    latb = jnp.repeat(lat, num_atoms, axis=0, total_repeat_length=N)
    cart = jnp.einsum('ni,nij->nj', frac_x, latb)
    cart16 = jnp.pad(cart, ((0, 0), (0, 13)))

    x0 = _tc_embed(h, zb, W_emb[:256], W_emb[256:], b_emb.reshape(1, -1))
    hblk = EH // BE

    Cs, Cd = _sc_gather2(cart16, src_full, dst_full, tiled=False)

    es, ms = [], []
    for hh in range(2):
        Xs, Xd = _sc_gather2(x0, srcs[hh], dsts[hh])
        e1h, m1h = _tc_pass1(Xs, Xd, Cs, Cd, hh * hblk,
                             W_edge, b_edge.reshape(1, -1),
                             W_msg[0], b_msg[0].reshape(1, -1))
        es.append(e1h)
        ms.append(m1h)
    P1 = [_sc_scatter(ms[hh], dst3s[hh]) for hh in range(2)]
    x1 = _tc_node(x0, [P1[0][0], P1[0][1], P1[1][0], P1[1][1]],
                  W_upd[0], b_upd[0].reshape(1, -1))

    e2s, m2s = [], []
    for hh in range(2):
        Xs, Xd = _sc_gather2(x1, srcs[hh], dsts[hh])
        e2h, m2h = _tc_pass2(es[hh], Xs, Xd, W_msg[1], b_msg[1].reshape(1, -1))
        e2s.append(e2h)
        m2s.append(m2h)
    P2 = [_sc_scatter(m2s[hh], dst3s[hh]) for hh in range(2)]
    x2 = _tc_node(x1, [P2[0][0], P2[0][1], P2[1][0], P2[1][1]],
                  W_upd[1], b_upd[1].reshape(1, -1))

    wox8 = jnp.pad(W_out_x, ((0, 0), (0, 7)))
    box8 = jnp.pad(b_out_x, (0, 7)).reshape(1, 8)
    m3s, uvs = [], []
    for hh in range(2):
        Xs, Xd = _sc_gather2(x2, srcs[hh], dsts[hh])
        m3h, uvh = _tc_pass3(e2s[hh], Xs, Xd, Cs, Cd, hh * hblk,
                             W_msg[2], b_msg[2].reshape(1, -1), wox8, box8)
        m3s.append(m3h)
        uvs.append(uvh)
    P3 = [_sc_scatter(m3s[hh], dst3s[hh]) for hh in range(2)]
    U = [_sc_scatter(uvs[hh], dst3s[hh], tiled=False) for hh in range(2)]
    x3, epsp = _tc_node3(x2, [P3[0][0], P3[0][1], P3[1][0], P3[1][1]],
                         [U[0][0], U[0][1], U[1][0], U[1][1]],
                         W_upd[2], b_upd[2].reshape(1, -1))

    pred_eps_h = _tc_outh(x3, W_out_h, b_out_h.reshape(1, -1))
    pred_eps_x = epsp[:, :3]
    return (pred_eps_x, pred_eps_h)


# speedup vs baseline: 4.4240x; 1.0242x over previous
import functools

import jax
import jax.numpy as jnp
from jax import lax
from jax.experimental import pallas as pl
from jax.experimental.pallas import tpu as pltpu
from jax.experimental.pallas import tpu_sc as plsc

N = 10000
E = 320000
HID = 128
NRBF = 16
NOUT = 100
CUTOFF = 6.0
WX = HID + 16

NC = 2
NS = 16
NW = NC * NS
CHUNK = 40
K = 5
EH = E // 2


def _silu(v):
    return v * jax.nn.sigmoid(v)


def _dot(a, b):
    return jnp.dot(a, b, preferred_element_type=jnp.float32)


def _sc_gather2(table, src2, dst2, tiled=True):
    W = table.shape[1]
    PER = src2.shape[1]
    NE = NW * PER
    NGROUP = PER // (K * CHUNK)
    mesh = plsc.VectorSubcoreMesh(core_axis_name="c", subcore_axis_name="s")
    params = (pltpu.CompilerParams() if tiled
              else pltpu.CompilerParams(use_tc_tiling_on_sc=False))

    @functools.partial(
        pl.kernel,
        out_type=[jax.ShapeDtypeStruct((NE, W), jnp.float32),
                  jax.ShapeDtypeStruct((NE, W), jnp.float32)],
        mesh=mesh,
        scratch_types=([pltpu.VMEM((PER,), jnp.int32)]
                       + [pltpu.VMEM((CHUNK, W), jnp.float32)
                          for _ in range(K)]
                       + [pltpu.SemaphoreType.DMA, pltpu.SemaphoreType.DMA,
                          pltpu.VMEM_SHARED((N, W), jnp.float32)]),
        compiler_params=params,
    )
    def k(tbl, s_idx, d_idx, o1, o2, idx1, r0, r1, r2, r3, r4,
          gsem, ssem, shtbl):
        cid = lax.axis_index("c")
        sid = lax.axis_index("s")
        wid = sid * NC + cid
        bufs = [r0, r1, r2, r3, r4]

        @pl.when(sid == 0)
        def _stage():
            pltpu.sync_copy(tbl, shtbl)
        plsc.subcore_barrier()

        def run(idx2_hbm, out_hbm):
            pltpu.sync_copy(idx2_hbm.at[wid], idx1)

            def group(g, carry):
                descs = [pltpu.async_copy(
                    shtbl.at[idx1.at[pl.ds((g * K + b) * CHUNK, CHUNK)]],
                    bufs[b], gsem)
                    for b in range(K)]
                for d in descs:
                    d.wait()
                base = wid * PER + g * (K * CHUNK)
                descs = [pltpu.async_copy(
                    bufs[b], out_hbm.at[pl.ds(base + b * CHUNK, CHUNK)], ssem)
                    for b in range(K)]
                for d in descs:
                    d.wait()
                return carry
            lax.fori_loop(0, NGROUP, group, 0)

        run(s_idx, o1)
        run(d_idx, o2)

    return k(table, src2, dst2)


def _sc_scatter(rows_in, dst3, tiled=True):
    W = rows_in.shape[1]
    PER = dst3.shape[1] * dst3.shape[2]
    NGROUP = PER // (K * CHUNK)
    mesh = plsc.VectorSubcoreMesh(core_axis_name="c", subcore_axis_name="s")
    zer = jnp.zeros((N, W), jnp.float32)
    params = (pltpu.CompilerParams() if tiled
              else pltpu.CompilerParams(use_tc_tiling_on_sc=False))

    @functools.partial(
        pl.kernel,
        out_type=jax.ShapeDtypeStruct((NC, N, W), jnp.float32),
        mesh=mesh,
        scratch_types=([pltpu.VMEM((K, CHUNK), jnp.int32)]
                       + [pltpu.VMEM((CHUNK, W), jnp.float32)
                          for _ in range(K)]
                       + [pltpu.SemaphoreType.DMA, pltpu.SemaphoreType.DMA,
                          pltpu.SemaphoreType.DMA,
                          pltpu.VMEM_SHARED((N, W), jnp.float32)]),
        compiler_params=params,
    )
    def k(rows_hbm, d_idx, zer_hbm, out, idxb, r0, r1, r2, r3, r4,
          isem, lsem, asem, agg):
        cid = lax.axis_index("c")
        sid = lax.axis_index("s")
        wid = sid * NC + cid
        bufs = [r0, r1, r2, r3, r4]

        @pl.when(sid == 0)
        def _zero():
            pltpu.sync_copy(zer_hbm, agg)
        plsc.subcore_barrier()

        def group(g, carry):
            base = wid * PER + g * (K * CHUNK)
            idescs = [pltpu.async_copy(
                d_idx.at[wid, g * K + b], idxb.at[b], isem)
                for b in range(K)]
            ldescs = [pltpu.async_copy(
                rows_hbm.at[pl.ds(base + b * CHUNK, CHUNK)], bufs[b], lsem)
                for b in range(K)]
            for d in idescs:
                d.wait()
            for d in ldescs:
                d.wait()
            descs = [pltpu.async_copy(
                bufs[b], agg.at[idxb.at[b]], asem, add=True)
                for b in range(K)]
            for d in descs:
                d.wait()
            return carry
        lax.fori_loop(0, NGROUP, group, 0)

        plsc.subcore_barrier()

        @pl.when(sid == 0)
        def _dump():
            pltpu.sync_copy(agg, out.at[cid])

    return k(rows_in, dst3, zer)


BN = 2000
BE = 4000


def _tc_embed(h, zb, W_h, W_z, b):
    def body(h_ref, z_ref, wh_ref, wz_ref, b_ref, o_ref):
        acc = _dot(h_ref[...], wh_ref[...]) + _dot(z_ref[...], wz_ref[...]) + b_ref[...]
        o_ref[...] = _silu(acc)

    return pl.pallas_call(
        body,
        grid=(N // BN,),
        in_specs=[pl.BlockSpec((BN, 256), lambda i: (i, 0)),
                  pl.BlockSpec((BN, 256), lambda i: (i, 0)),
                  pl.BlockSpec((256, HID), lambda i: (0, 0)),
                  pl.BlockSpec((256, HID), lambda i: (0, 0)),
                  pl.BlockSpec((1, HID), lambda i: (0, 0))],
        out_specs=pl.BlockSpec((BN, HID), lambda i: (i, 0)),
        out_shape=jax.ShapeDtypeStruct((N, HID), jnp.float32),
    )(h, zb, W_h, W_z, b)


def _tc_pass1(Xs, Xd, Cs, Cd, hoff, We, be, W1, b1):
    ne = Xs.shape[0]
    def body(xs_ref, xd_ref, cs_ref, cd_ref, we_ref, be_ref, w1_ref, b1_ref,
             e1_ref, m1_ref):
        xs = xs_ref[...]
        xd = xd_ref[...]
        dv = cd_ref[...] - cs_ref[...]
        dist = jnp.sqrt(jnp.sum(dv * dv, axis=1, keepdims=True) + 1e-9)
        centers = (CUTOFF / (NRBF - 1)) * lax.broadcasted_iota(
            jnp.int32, (1, NRBF), 1).astype(jnp.float32)
        diff = dist - centers
        rbf = jnp.exp(-10.0 * diff * diff)
        pre = (_dot(xs, we_ref[:HID]) + _dot(xd, we_ref[HID:2 * HID])
               + _dot(rbf, we_ref[2 * HID:]) + be_ref[...])
        e0 = _silu(pre)
        t = e0 + xs + xd
        m1 = _silu(_dot(t, w1_ref[...]) + b1_ref[...])
        e1_ref[...] = (e0 + m1).astype(jnp.bfloat16)
        m1_ref[...] = m1

    return pl.pallas_call(
        body,
        grid=(ne // BE,),
        in_specs=[pl.BlockSpec((BE, HID), lambda i: (i, 0)),
                  pl.BlockSpec((BE, HID), lambda i: (i, 0)),
                  pl.BlockSpec((BE, 16), lambda i: (i + hoff, 0)),
                  pl.BlockSpec((BE, 16), lambda i: (i + hoff, 0)),
                  pl.BlockSpec((2 * HID + NRBF, HID), lambda i: (0, 0)),
                  pl.BlockSpec((1, HID), lambda i: (0, 0)),
                  pl.BlockSpec((HID, HID), lambda i: (0, 0)),
                  pl.BlockSpec((1, HID), lambda i: (0, 0))],
        out_specs=[pl.BlockSpec((BE, HID), lambda i: (i, 0)),
                   pl.BlockSpec((BE, HID), lambda i: (i, 0))],
        out_shape=[jax.ShapeDtypeStruct((ne, HID), jnp.bfloat16),
                   jax.ShapeDtypeStruct((ne, HID), jnp.float32)],
    )(Xs, Xd, Cs, Cd, We, be, W1, b1)


def _tc_pass2(e, Xs, Xd, W, b):
    ne = Xs.shape[0]

    def body(e_ref, xs_ref, xd_ref, w_ref, b_ref, e2_ref, m_ref):
        ef = e_ref[...].astype(jnp.float32)
        t = ef + xs_ref[...] + xd_ref[...]
        m = _silu(_dot(t, w_ref[...]) + b_ref[...])
        m_ref[...] = m
        e2_ref[...] = (ef + m).astype(jnp.bfloat16)

    return pl.pallas_call(
        body,
        grid=(ne // BE,),
        in_specs=[pl.BlockSpec((BE, HID), lambda i: (i, 0)),
                  pl.BlockSpec((BE, HID), lambda i: (i, 0)),
                  pl.BlockSpec((BE, HID), lambda i: (i, 0)),
                  pl.BlockSpec((HID, HID), lambda i: (0, 0)),
                  pl.BlockSpec((1, HID), lambda i: (0, 0))],
        out_specs=[pl.BlockSpec((BE, HID), lambda i: (i, 0)),
                   pl.BlockSpec((BE, HID), lambda i: (i, 0))],
        out_shape=[jax.ShapeDtypeStruct((ne, HID), jnp.bfloat16),
                   jax.ShapeDtypeStruct((ne, HID), jnp.float32)],
    )(e, Xs, Xd, W, b)


def _tc_pass3(e, Xs, Xd, Cs, Cd, hoff, W, b, wox8, box8):
    ne = Xs.shape[0]

    def body(e_ref, xs_ref, xd_ref, cs_ref, cd_ref, w_ref, b_ref,
             wox_ref, box_ref, m_ref, uv_ref):
        ef = e_ref[...].astype(jnp.float32)
        t = ef + xs_ref[...] + xd_ref[...]
        m = _silu(_dot(t, w_ref[...]) + b_ref[...])
        e3 = ef + m
        scal = _dot(e3, wox_ref[...]) + box_ref[...]
        m_ref[...] = m
        dv = cd_ref[...] - cs_ref[...]
        dist = jnp.sqrt(jnp.sum(dv * dv, axis=1, keepdims=True) + 1e-9)
        uv_ref[...] = (scal[:, 0:1] / dist) * dv

    return pl.pallas_call(
        body,
        grid=(ne // BE,),
        in_specs=[pl.BlockSpec((BE, HID), lambda i: (i, 0)),
                  pl.BlockSpec((BE, HID), lambda i: (i, 0)),
                  pl.BlockSpec((BE, HID), lambda i: (i, 0)),
                  pl.BlockSpec((BE, 16), lambda i: (i + hoff, 0)),
                  pl.BlockSpec((BE, 16), lambda i: (i + hoff, 0)),
                  pl.BlockSpec((HID, HID), lambda i: (0, 0)),
                  pl.BlockSpec((1, HID), lambda i: (0, 0)),
                  pl.BlockSpec((HID, 8), lambda i: (0, 0)),
                  pl.BlockSpec((1, 8), lambda i: (0, 0))],
        out_specs=[pl.BlockSpec((BE, HID), lambda i: (i, 0)),
                   pl.BlockSpec((BE, 16), lambda i: (i, 0))],
        out_shape=[jax.ShapeDtypeStruct((ne, HID), jnp.float32),
                   jax.ShapeDtypeStruct((ne, 16), jnp.float32)],
    )(e, Xs, Xd, Cs, Cd, W, b, wox8, box8)


def _tc_node(x, aggs, W, b):
    def body(x_ref, a0_ref, a1_ref, a2_ref, a3_ref, w_ref, b_ref, o_ref):
        agg = (a0_ref[...] + a1_ref[...]) + (a2_ref[...] + a3_ref[...])
        o_ref[...] = x_ref[...] + _silu(_dot(agg, w_ref[...]) + b_ref[...])

    return pl.pallas_call(
        body,
        grid=(N // BN,),
        in_specs=[pl.BlockSpec((BN, HID), lambda i: (i, 0))]
                 + [pl.BlockSpec((BN, HID), lambda i: (i, 0))] * 4
                 + [pl.BlockSpec((HID, HID), lambda i: (0, 0)),
                    pl.BlockSpec((1, HID), lambda i: (0, 0))],
        out_specs=pl.BlockSpec((BN, HID), lambda i: (i, 0)),
        out_shape=jax.ShapeDtypeStruct((N, HID), jnp.float32),
    )(x, *aggs, W, b)


def _tc_node3(x, aggs, us, W, b):
    def body(x_ref, a0_ref, a1_ref, a2_ref, a3_ref,
             u0_ref, u1_ref, u2_ref, u3_ref, w_ref, b_ref,
             x3_ref, eps_ref):
        agg = (a0_ref[...] + a1_ref[...]) + (a2_ref[...] + a3_ref[...])
        x3_ref[...] = x_ref[...] + _silu(_dot(agg, w_ref[...]) + b_ref[...])
        eps_ref[...] = ((u0_ref[...] + u1_ref[...])
                        + (u2_ref[...] + u3_ref[...]))

    return pl.pallas_call(
        body,
        grid=(N // BN,),
        in_specs=[pl.BlockSpec((BN, HID), lambda i: (i, 0))]
                 + [pl.BlockSpec((BN, HID), lambda i: (i, 0))] * 4
                 + [pl.BlockSpec((BN, 16), lambda i: (i, 0))] * 4
                 + [pl.BlockSpec((HID, HID), lambda i: (0, 0)),
                    pl.BlockSpec((1, HID), lambda i: (0, 0))],
        out_specs=[pl.BlockSpec((BN, HID), lambda i: (i, 0)),
                   pl.BlockSpec((BN, 16), lambda i: (i, 0))],
        out_shape=[jax.ShapeDtypeStruct((N, HID), jnp.float32),
                   jax.ShapeDtypeStruct((N, 16), jnp.float32)],
    )(x, *aggs, *us, W, b)


def _tc_outh(x, W, b):
    def body(x_ref, w_ref, b_ref, o_ref):
        o_ref[...] = _dot(x_ref[...], w_ref[...]) + b_ref[...]

    return pl.pallas_call(
        body,
        grid=(N // BN,),
        in_specs=[pl.BlockSpec((BN, HID), lambda i: (i, 0)),
                  pl.BlockSpec((HID, NOUT), lambda i: (0, 0)),
                  pl.BlockSpec((1, NOUT), lambda i: (0, 0))],
        out_specs=pl.BlockSpec((BN, NOUT), lambda i: (i, 0)),
        out_shape=jax.ShapeDtypeStruct((N, NOUT), jnp.float32),
    )(x, W, b)


def _lattice(lengths, angles):
    a, b, c = lengths[:, 0], lengths[:, 1], lengths[:, 2]
    al = jnp.deg2rad(angles[:, 0])
    be = jnp.deg2rad(angles[:, 1])
    ga = jnp.deg2rad(angles[:, 2])
    cos_a, cos_b, cos_g = jnp.cos(al), jnp.cos(be), jnp.cos(ga)
    sin_g = jnp.sin(ga)
    zeros = jnp.zeros_like(a)
    v1 = jnp.stack([a, zeros, zeros], -1)
    v2 = jnp.stack([b * cos_g, b * sin_g, zeros], -1)
    cx = cos_b
    cy = (cos_a - cos_b * cos_g) / sin_g
    cz = jnp.sqrt(jnp.clip(1.0 - cx ** 2 - cy ** 2, 1e-8, None))
    v3 = jnp.stack([c * cx, c * cy, c * cz], -1)
    return jnp.stack([v1, v2, v3], 1)


def kernel(z, frac_x, h, num_atoms, lengths, angles, edge_index,
           W_emb, b_emb, W_edge, b_edge, W_msg, b_msg, W_upd, b_upd,
           W_out_h, b_out_h, W_out_x, b_out_x):
    perh = EH // NW
    srcs, dsts, dst3s = [], [], []
    for hh in range(2):
        sl = slice(hh * EH, (hh + 1) * EH)
        srcs.append(edge_index[0, sl].reshape(NW, perh))
        dsts.append(edge_index[1, sl].reshape(NW, perh))
        dst3s.append(edge_index[1, sl].reshape(NW, perh // CHUNK, CHUNK))
    src_full = edge_index[0].reshape(NW, E // NW)
    dst_full = edge_index[1].reshape(NW, E // NW)

    zb = jnp.repeat(z, num_atoms, axis=0, total_repeat_length=N)
    lat = _lattice(lengths, angles)
    latb = jnp.repeat(lat, num_atoms, axis=0, total_repeat_length=N)
    cart = jnp.einsum('ni,nij->nj', frac_x, latb)
    cart16 = jnp.pad(cart, ((0, 0), (0, 13)))

    x0 = _tc_embed(h, zb, W_emb[:256], W_emb[256:], b_emb.reshape(1, -1))
    hblk = EH // BE

    Cs, Cd = _sc_gather2(cart16, src_full, dst_full, tiled=False)

    es, ms = [], []
    for hh in range(2):
        Xs, Xd = _sc_gather2(x0, srcs[hh], dsts[hh])
        e1h, m1h = _tc_pass1(Xs, Xd, Cs, Cd, hh * hblk,
                             W_edge, b_edge.reshape(1, -1),
                             W_msg[0], b_msg[0].reshape(1, -1))
        es.append(e1h)
        ms.append(m1h)
    P1 = [_sc_scatter(ms[hh], dst3s[hh]) for hh in range(2)]
    x1 = _tc_node(x0, [P1[0][0], P1[0][1], P1[1][0], P1[1][1]],
                  W_upd[0], b_upd[0].reshape(1, -1))

    e2s, m2s = [], []
    for hh in range(2):
        Xs, Xd = _sc_gather2(x1, srcs[hh], dsts[hh])
        e2h, m2h = _tc_pass2(es[hh], Xs, Xd, W_msg[1], b_msg[1].reshape(1, -1))
        e2s.append(e2h)
        m2s.append(m2h)
    P2 = [_sc_scatter(m2s[hh], dst3s[hh]) for hh in range(2)]
    x2 = _tc_node(x1, [P2[0][0], P2[0][1], P2[1][0], P2[1][1]],
                  W_upd[1], b_upd[1].reshape(1, -1))

    wox8 = jnp.pad(W_out_x, ((0, 0), (0, 7)))
    box8 = jnp.pad(b_out_x, (0, 7)).reshape(1, 8)
    m3s, uvs = [], []
    for hh in range(2):
        Xs, Xd = _sc_gather2(x2, srcs[hh], dsts[hh])
        m3h, uvh = _tc_pass3(e2s[hh], Xs, Xd, Cs, Cd, hh * hblk,
                             W_msg[2], b_msg[2].reshape(1, -1), wox8, box8)
        m3s.append(m3h)
        uvs.append(uvh)
    P3 = [_sc_scatter(m3s[hh], dst3s[hh]) for hh in range(2)]
    U = [_sc_scatter(uvs[hh], dst3s[hh], tiled=False) for hh in range(2)]
    x3, epsp = _tc_node3(x2, [P3[0][0], P3[0][1], P3[1][0], P3[1][1]],
                         [U[0][0], U[0][1], U[1][0], U[1][1]],
                         W_upd[2], b_upd[2].reshape(1, -1))

    pred_eps_h = _tc_outh(x3, W_out_h, b_out_h.reshape(1, -1))
    pred_eps_x = epsp[:, :3]
    return (pred_eps_x, pred_eps_h)
